# Initial kernel scaffold; baseline (speedup 1.0000x reference)
import jax
import jax.numpy as jnp
from jax.experimental import pallas as pl


def kernel(x, edge_index, edge_attr, W1, b1, W2, b2):
    raise NotImplementedError("write your pallas kernel here")


# speedup vs baseline: 12.5867x; 12.5867x over previous
import functools

import jax
import jax.numpy as jnp
from jax import lax
from jax.experimental import pallas as pl
from jax.experimental.pallas import tpu as pltpu
from jax.experimental.pallas import tpu_sc as plsc

N = 10000
D = 128
E = 320000

NP = 10240
NW = 32
CHUNK = 128
EP_W = 10240
NCH = EP_W // CHUNK
EP = EP_W * NW
ROWS_T = NP // 16

_mesh2 = plsc.VectorSubcoreMesh(core_axis_name="c", subcore_axis_name="s",
                                num_cores=2)
_mesh1 = plsc.VectorSubcoreMesh(core_axis_name="c", subcore_axis_name="s",
                                num_cores=1)


@functools.partial(
    pl.kernel,
    mesh=_mesh2,
    compiler_params=pltpu.CompilerParams(needs_layout_passes=False),
    out_type=jax.ShapeDtypeStruct((2, NP), jnp.float32),
    scratch_types=[
        pltpu.VMEM((NCH, CHUNK), jnp.int32),
        pltpu.VMEM((EP_W,), jnp.float32),
        pltpu.VMEM((ROWS_T,), jnp.float32),
        pltpu.VMEM_SHARED((NP,), jnp.float32),
    ],
)
def _sc_deg(dst_hbm, w_hbm, dp_hbm, dst_v, w_v, stage_v, acc):
    c = lax.axis_index("c")
    s = lax.axis_index("s")
    wid = c * 16 + s

    def _init(i, _):
        stage_v[pl.ds(16 * i, 16)] = jnp.full((16,), 0.5, jnp.float32)
        return 0
    lax.fori_loop(0, ROWS_T // 16, _init, 0)
    pltpu.sync_copy(stage_v, acc.at[pl.ds(ROWS_T * s, ROWS_T)])
    plsc.subcore_barrier()

    pltpu.sync_copy(dst_hbm.at[pl.ds(wid * NCH, NCH)], dst_v)
    pltpu.sync_copy(w_hbm.at[pl.ds(wid * EP_W, EP_W)], w_v)

    def _chunk(j, _):
        pltpu.sync_copy(w_v.at[pl.ds(j * CHUNK, CHUNK)],
                        acc.at[dst_v.at[j]], add=True)
        return 0
    lax.fori_loop(0, NCH, _chunk, 0)
    plsc.subcore_barrier()

    pltpu.sync_copy(acc.at[pl.ds(ROWS_T * s, ROWS_T)],
                    dp_hbm.at[c, pl.ds(ROWS_T * s, ROWS_T)])


def _tc_b_body(x_ref, w1_ref, d0_ref, d1_ref, h1s_ref, dis_ref):
    deg = d0_ref[...] + d1_ref[...]
    dis = lax.rsqrt(deg)
    h1s_ref[...] = jnp.dot(x_ref[...], w1_ref[...],
                           preferred_element_type=jnp.float32) * dis
    dis_ref[...] = dis


def _tc_b(x_p, W1, d0, d1):
    grid = (NP // 128,)
    return pl.pallas_call(
        _tc_b_body,
        grid=grid,
        in_specs=[
            pl.BlockSpec((128, D), lambda i: (i, 0)),
            pl.BlockSpec((D, D), lambda i: (0, 0)),
            pl.BlockSpec((128, 1), lambda i: (i, 0)),
            pl.BlockSpec((128, 1), lambda i: (i, 0)),
        ],
        out_specs=[
            pl.BlockSpec((128, D), lambda i: (i, 0)),
            pl.BlockSpec((128, 1), lambda i: (i, 0)),
        ],
        out_shape=[
            jax.ShapeDtypeStruct((NP, D), jnp.float32),
            jax.ShapeDtypeStruct((NP, 1), jnp.float32),
        ],
    )(x_p, W1, d0, d1)


@functools.partial(
    pl.kernel,
    mesh=_mesh2,
    compiler_params=pltpu.CompilerParams(needs_layout_passes=False),
    out_type=jax.ShapeDtypeStruct((2, NP, D), jnp.float32),
    scratch_types=[
        pltpu.VMEM((NCH, CHUNK), jnp.int32),
        pltpu.VMEM((NCH, CHUNK), jnp.int32),
        pltpu.VMEM((EP_W,), jnp.float32),
        pltpu.VMEM((CHUNK, D), jnp.float32),
        pltpu.VMEM_SHARED((NP, D), jnp.float32),
    ],
)
def _sc_rows(src_hbm, dst_hbm, w_hbm, h1s_hbm, p_hbm,
             src_v, dst_v, w_v, rows_v, acc):
    c = lax.axis_index("c")
    s = lax.axis_index("s")
    wid = c * 16 + s

    def _zrow(i, _):
        for k in range(D // 16):
            rows_v[i, pl.ds(16 * k, 16)] = jnp.zeros((16,), jnp.float32)
        return 0
    lax.fori_loop(0, CHUNK, _zrow, 0)
    for m in range(ROWS_T // CHUNK):
        pltpu.sync_copy(rows_v, acc.at[pl.ds(ROWS_T * s + CHUNK * m, CHUNK)])
    plsc.subcore_barrier()

    pltpu.sync_copy(src_hbm.at[pl.ds(wid * NCH, NCH)], src_v)
    pltpu.sync_copy(dst_hbm.at[pl.ds(wid * NCH, NCH)], dst_v)
    pltpu.sync_copy(w_hbm.at[pl.ds(wid * EP_W, EP_W)], w_v)

    def _chunk(j, _):
        pltpu.sync_copy(h1s_hbm.at[src_v.at[j]], rows_v)

        def _edge(e, _):
            wv = plsc.load_gather(w_v, [jnp.full((16,), j * CHUNK + e,
                                                 jnp.int32)])
            for k in range(D // 16):
                rows_v[e, pl.ds(16 * k, 16)] = rows_v[e, pl.ds(16 * k, 16)] * wv
            return 0
        lax.fori_loop(0, CHUNK, _edge, 0)
        pltpu.sync_copy(rows_v, acc.at[dst_v.at[j]], add=True)
        return 0
    lax.fori_loop(0, NCH, _chunk, 0)
    plsc.subcore_barrier()

    for m in range(ROWS_T // CHUNK):
        r = ROWS_T * s + CHUNK * m
        pltpu.sync_copy(acc.at[pl.ds(r, CHUNK)], p_hbm.at[c, pl.ds(r, CHUNK)])


def _tc_d_body(p0_ref, p1_ref, h1s_ref, dis_ref, b1_ref, w2_ref,
               emb_ref, h2s_ref):
    dis = dis_ref[...]
    s = dis * (p0_ref[...] + p1_ref[...] + h1s_ref[...]) + b1_ref[...]
    h = jnp.where(s > 0, s, jnp.exp(s) - 1.0)
    emb_ref[...] = h
    h2s_ref[...] = jnp.dot(h * dis, w2_ref[...],
                           preferred_element_type=jnp.float32)


def _tc_d(p0, p1, h1s, dis2, b1r, W2):
    grid = (NP // 128,)
    return pl.pallas_call(
        _tc_d_body,
        grid=grid,
        in_specs=[
            pl.BlockSpec((128, D), lambda i: (i, 0)),
            pl.BlockSpec((128, D), lambda i: (i, 0)),
            pl.BlockSpec((128, D), lambda i: (i, 0)),
            pl.BlockSpec((128, 1), lambda i: (i, 0)),
            pl.BlockSpec((1, D), lambda i: (0, 0)),
            pl.BlockSpec((D, 1), lambda i: (0, 0)),
        ],
        out_specs=[
            pl.BlockSpec((128, D), lambda i: (i, 0)),
            pl.BlockSpec((128, 1), lambda i: (i, 0)),
        ],
        out_shape=[
            jax.ShapeDtypeStruct((NP, D), jnp.float32),
            jax.ShapeDtypeStruct((NP, 1), jnp.float32),
        ],
    )(p0, p1, h1s, dis2, b1r, W2)


EP_W1 = EP // 16
NCH1 = EP_W1 // CHUNK


@functools.partial(
    pl.kernel,
    mesh=_mesh1,
    compiler_params=pltpu.CompilerParams(needs_layout_passes=False),
    out_type=jax.ShapeDtypeStruct((NP,), jnp.float32),
    scratch_types=[
        pltpu.VMEM((NCH1, CHUNK), jnp.int32),
        pltpu.VMEM((NCH1, CHUNK), jnp.int32),
        pltpu.VMEM((EP_W1,), jnp.float32),
        pltpu.VMEM((NP,), jnp.float32),
        pltpu.VMEM((CHUNK,), jnp.float32),
        pltpu.VMEM((ROWS_T,), jnp.float32),
        pltpu.VMEM((ROWS_T,), jnp.float32),
        pltpu.VMEM((ROWS_T,), jnp.float32),
        pltpu.VMEM((16,), jnp.float32),
        pltpu.VMEM_SHARED((NP,), jnp.float32),
    ],
)
def _sc_l2(src_hbm, dst_hbm, w_hbm, h2s_hbm, dis_hbm, b2_hbm, out_hbm,
           src_v, dst_v, w_v, h2s_v, msg_v, a_v, dis_v, o_v, b2_v, acc):
    s = lax.axis_index("s")

    def _init(i, _):
        o_v[pl.ds(16 * i, 16)] = jnp.zeros((16,), jnp.float32)
        return 0
    lax.fori_loop(0, ROWS_T // 16, _init, 0)
    pltpu.sync_copy(o_v, acc.at[pl.ds(ROWS_T * s, ROWS_T)])
    plsc.subcore_barrier()

    pltpu.sync_copy(src_hbm.at[pl.ds(s * NCH1, NCH1)], src_v)
    pltpu.sync_copy(dst_hbm.at[pl.ds(s * NCH1, NCH1)], dst_v)
    pltpu.sync_copy(w_hbm.at[pl.ds(s * EP_W1, EP_W1)], w_v)
    pltpu.sync_copy(h2s_hbm, h2s_v)

    def _chunk(j, _):
        for m in range(CHUNK // 16):
            idx = src_v[j, pl.ds(16 * m, 16)]
            hv = plsc.load_gather(h2s_v, [idx])
            wv = w_v[pl.ds(j * CHUNK + 16 * m, 16)]
            msg_v[pl.ds(16 * m, 16)] = hv * wv
        pltpu.sync_copy(msg_v, acc.at[dst_v.at[j]], add=True)
        return 0
    lax.fori_loop(0, NCH1, _chunk, 0)
    plsc.subcore_barrier()

    pltpu.sync_copy(acc.at[pl.ds(ROWS_T * s, ROWS_T)], a_v)
    pltpu.sync_copy(dis_hbm.at[pl.ds(ROWS_T * s, ROWS_T)], dis_v)
    pltpu.sync_copy(b2_hbm, b2_v)
    bv = b2_v[...]

    def _node(i, _):
        sl = pl.ds(16 * i, 16)
        hv = h2s_v[pl.ds(ROWS_T * s + 16 * i, 16)]
        z = dis_v[sl] * (a_v[sl] + hv) + bv
        o_v[sl] = 1.0 / (1.0 + jnp.exp(-z))
        return 0
    lax.fori_loop(0, ROWS_T // 16, _node, 0)
    pltpu.sync_copy(o_v, out_hbm.at[pl.ds(ROWS_T * s, ROWS_T)])


def kernel(x, edge_index, edge_attr, W1, b1, W2, b2):
    x_p = jnp.pad(x, ((0, NP - N), (0, 0)))
    src = jnp.pad(edge_index[0], (0, EP - E)).reshape(NW * NCH, CHUNK)
    dst = jnp.pad(edge_index[1], (0, EP - E)).reshape(NW * NCH, CHUNK)
    w_p = jnp.pad(edge_attr, (0, EP - E))

    dp = _sc_deg(dst, w_p)
    h1s, dis2 = _tc_b(x_p, W1, dp[0][:, None], dp[1][:, None])
    p = _sc_rows(src, dst, w_p, h1s)
    emb_p, h2s2 = _tc_d(p[0], p[1], h1s, dis2, b1[None, :], W2)
    b2v = jnp.broadcast_to(b2, (16,)).astype(jnp.float32)
    out_flat = _sc_l2(src, dst, w_p, h2s2[:, 0], dis2[:, 0], b2v)
    return (out_flat[:N, None], emb_p[:N])


# speedup vs baseline: 12.5906x; 1.0003x over previous
import functools

import jax
import jax.numpy as jnp
from jax import lax
from jax.experimental import pallas as pl
from jax.experimental.pallas import tpu as pltpu
from jax.experimental.pallas import tpu_sc as plsc

N = 10000
D = 128
E = 320000

NP = 10240
NW = 32
CHUNK = 128
EP_W = 10240
NCH = EP_W // CHUNK
EP = EP_W * NW
ROWS_T = NP // 16

_mesh2 = plsc.VectorSubcoreMesh(core_axis_name="c", subcore_axis_name="s",
                                num_cores=2)
_mesh1 = plsc.VectorSubcoreMesh(core_axis_name="c", subcore_axis_name="s",
                                num_cores=1)


@functools.partial(
    pl.kernel,
    mesh=_mesh2,
    compiler_params=pltpu.CompilerParams(needs_layout_passes=False),
    out_type=jax.ShapeDtypeStruct((2, NP), jnp.float32),
    scratch_types=[
        pltpu.VMEM((NCH, CHUNK), jnp.int32),
        pltpu.VMEM((EP_W,), jnp.float32),
        pltpu.VMEM((ROWS_T,), jnp.float32),
        pltpu.VMEM_SHARED((NP,), jnp.float32),
    ],
)
def _sc_deg(dst_hbm, w_hbm, dp_hbm, dst_v, w_v, stage_v, acc):
    c = lax.axis_index("c")
    s = lax.axis_index("s")
    wid = c * 16 + s

    def _init(i, _):
        stage_v[pl.ds(16 * i, 16)] = jnp.full((16,), 0.5, jnp.float32)
        return 0
    lax.fori_loop(0, ROWS_T // 16, _init, 0)
    pltpu.sync_copy(stage_v, acc.at[pl.ds(ROWS_T * s, ROWS_T)])
    plsc.subcore_barrier()

    pltpu.sync_copy(dst_hbm.at[pl.ds(wid * NCH, NCH)], dst_v)
    pltpu.sync_copy(w_hbm.at[pl.ds(wid * EP_W, EP_W)], w_v)

    def _chunk(j, _):
        pltpu.sync_copy(w_v.at[pl.ds(j * CHUNK, CHUNK)],
                        acc.at[dst_v.at[j]], add=True)
        return 0
    lax.fori_loop(0, NCH, _chunk, 0)
    plsc.subcore_barrier()

    pltpu.sync_copy(acc.at[pl.ds(ROWS_T * s, ROWS_T)],
                    dp_hbm.at[c, pl.ds(ROWS_T * s, ROWS_T)])


def _tc_b_body(x_ref, w1_ref, d0_ref, d1_ref, h1s_ref, dis_ref):
    deg = d0_ref[...] + d1_ref[...]
    dis = lax.rsqrt(deg)
    h1s_ref[...] = jnp.dot(x_ref[...], w1_ref[...],
                           preferred_element_type=jnp.float32) * dis
    dis_ref[...] = dis


def _tc_b(x_p, W1, d0, d1):
    grid = (NP // 128,)
    return pl.pallas_call(
        _tc_b_body,
        grid=grid,
        in_specs=[
            pl.BlockSpec((128, D), lambda i: (i, 0)),
            pl.BlockSpec((D, D), lambda i: (0, 0)),
            pl.BlockSpec((128, 1), lambda i: (i, 0)),
            pl.BlockSpec((128, 1), lambda i: (i, 0)),
        ],
        out_specs=[
            pl.BlockSpec((128, D), lambda i: (i, 0)),
            pl.BlockSpec((128, 1), lambda i: (i, 0)),
        ],
        out_shape=[
            jax.ShapeDtypeStruct((NP, D), jnp.float32),
            jax.ShapeDtypeStruct((NP, 1), jnp.float32),
        ],
    )(x_p, W1, d0, d1)


CH_C = 64
NCH_C = EP_W // CH_C


@functools.partial(
    pl.kernel,
    mesh=_mesh2,
    compiler_params=pltpu.CompilerParams(needs_layout_passes=False),
    out_type=jax.ShapeDtypeStruct((2, NP, D), jnp.float32),
    scratch_types=[
        pltpu.VMEM((EP_W,), jnp.int32),
        pltpu.VMEM((NCH_C, CH_C), jnp.int32),
        pltpu.VMEM((CH_C,), jnp.float32),
        pltpu.VMEM((CH_C,), jnp.float32),
        pltpu.VMEM((CH_C, D), jnp.float32),
        pltpu.VMEM((CH_C, D), jnp.float32),
        pltpu.SemaphoreType.DMA,
        pltpu.SemaphoreType.DMA,
        pltpu.SemaphoreType.DMA,
        pltpu.SemaphoreType.DMA,
        pltpu.SemaphoreType.DMA,
        pltpu.SemaphoreType.DMA,
        pltpu.VMEM_SHARED((NP, D), jnp.float32),
    ],
)
def _sc_rows(src_hbm, dst_hbm, w_hbm, h1s_hbm, p_hbm,
             src_v, dst_v, wb0, wb1, rows0, rows1,
             g0, g1, gw0, gw1, s0, s1, acc):
    c = lax.axis_index("c")
    s = lax.axis_index("s")
    wid = c * 16 + s

    def _zrow(i, _):
        for k in range(D // 16):
            rows0[i, pl.ds(16 * k, 16)] = jnp.zeros((16,), jnp.float32)
        return 0
    lax.fori_loop(0, CH_C, _zrow, 0)
    for m in range(ROWS_T // CH_C):
        pltpu.sync_copy(rows0, acc.at[pl.ds(ROWS_T * s + CH_C * m, CH_C)])
    plsc.subcore_barrier()

    pltpu.sync_copy(src_hbm.at[pl.ds(wid * EP_W, EP_W)], src_v)
    pltpu.sync_copy(dst_hbm.at[pl.ds(wid * NCH_C, NCH_C)], dst_v)

    def _issue(n, rowsb, wbuf, gs, gws):
        pltpu.async_copy(h1s_hbm.at[src_v.at[pl.ds(n * CH_C, CH_C)]],
                         rowsb, gs)
        pltpu.async_copy(w_hbm.at[pl.ds(wid * EP_W + n * CH_C, CH_C)],
                         wbuf, gws)

    def _wait(n, rowsb, wbuf, gs, gws):
        pltpu.make_async_copy(h1s_hbm.at[src_v.at[pl.ds(n * CH_C, CH_C)]],
                              rowsb, gs).wait()
        pltpu.make_async_copy(w_hbm.at[pl.ds(wid * EP_W + n * CH_C, CH_C)],
                              wbuf, gws).wait()

    def _scale(buf, wbuf):
        def _edge(e, _):
            wv = plsc.load_gather(wbuf, [jnp.full((16,), e, jnp.int32)])
            for k in range(D // 16):
                buf[e, pl.ds(16 * k, 16)] = buf[e, pl.ds(16 * k, 16)] * wv
            return 0
        lax.fori_loop(0, CH_C, _edge, 0, unroll=4)

    _issue(0, rows0, wb0, g0, gw0)
    _issue(1, rows1, wb1, g1, gw1)

    def _pair(t, _):
        c0 = 2 * t
        c1 = 2 * t + 1
        _wait(c0, rows0, wb0, g0, gw0)
        _scale(rows0, wb0)
        cp0 = pltpu.async_copy(rows0, acc.at[dst_v.at[c0]], s0, add=True)
        _wait(c1, rows1, wb1, g1, gw1)
        _scale(rows1, wb1)
        cp1 = pltpu.async_copy(rows1, acc.at[dst_v.at[c1]], s1, add=True)
        n0 = jnp.minimum(c0 + 2, NCH_C - 1)
        n1 = jnp.minimum(c1 + 2, NCH_C - 1)
        cp0.wait()
        _issue(n0, rows0, wb0, g0, gw0)
        cp1.wait()
        _issue(n1, rows1, wb1, g1, gw1)
        return 0
    lax.fori_loop(0, NCH_C // 2, _pair, 0)
    _wait(NCH_C - 1, rows0, wb0, g0, gw0)
    _wait(NCH_C - 1, rows1, wb1, g1, gw1)
    plsc.subcore_barrier()

    for m in range(ROWS_T // CH_C):
        r = ROWS_T * s + CH_C * m
        pltpu.sync_copy(acc.at[pl.ds(r, CH_C)], p_hbm.at[c, pl.ds(r, CH_C)])


def _tc_d_body(p0_ref, p1_ref, h1s_ref, dis_ref, b1_ref, w2_ref,
               emb_ref, h2s_ref):
    dis = dis_ref[...]
    s = dis * (p0_ref[...] + p1_ref[...] + h1s_ref[...]) + b1_ref[...]
    h = jnp.where(s > 0, s, jnp.exp(s) - 1.0)
    emb_ref[...] = h
    h2s_ref[...] = jnp.dot(h * dis, w2_ref[...],
                           preferred_element_type=jnp.float32)


def _tc_d(p0, p1, h1s, dis2, b1r, W2):
    grid = (NP // 128,)
    return pl.pallas_call(
        _tc_d_body,
        grid=grid,
        in_specs=[
            pl.BlockSpec((128, D), lambda i: (i, 0)),
            pl.BlockSpec((128, D), lambda i: (i, 0)),
            pl.BlockSpec((128, D), lambda i: (i, 0)),
            pl.BlockSpec((128, 1), lambda i: (i, 0)),
            pl.BlockSpec((1, D), lambda i: (0, 0)),
            pl.BlockSpec((D, 1), lambda i: (0, 0)),
        ],
        out_specs=[
            pl.BlockSpec((128, D), lambda i: (i, 0)),
            pl.BlockSpec((128, 1), lambda i: (i, 0)),
        ],
        out_shape=[
            jax.ShapeDtypeStruct((NP, D), jnp.float32),
            jax.ShapeDtypeStruct((NP, 1), jnp.float32),
        ],
    )(p0, p1, h1s, dis2, b1r, W2)


EP_W1 = EP // 16
NCH1 = EP_W1 // CHUNK


@functools.partial(
    pl.kernel,
    mesh=_mesh1,
    compiler_params=pltpu.CompilerParams(needs_layout_passes=False),
    out_type=jax.ShapeDtypeStruct((NP,), jnp.float32),
    scratch_types=[
        pltpu.VMEM((NCH1, CHUNK), jnp.int32),
        pltpu.VMEM((NCH1, CHUNK), jnp.int32),
        pltpu.VMEM((EP_W1,), jnp.float32),
        pltpu.VMEM((NP,), jnp.float32),
        pltpu.VMEM((CHUNK,), jnp.float32),
        pltpu.VMEM((ROWS_T,), jnp.float32),
        pltpu.VMEM((ROWS_T,), jnp.float32),
        pltpu.VMEM((ROWS_T,), jnp.float32),
        pltpu.VMEM((16,), jnp.float32),
        pltpu.VMEM_SHARED((NP,), jnp.float32),
    ],
)
def _sc_l2(src_hbm, dst_hbm, w_hbm, h2s_hbm, dis_hbm, b2_hbm, out_hbm,
           src_v, dst_v, w_v, h2s_v, msg_v, a_v, dis_v, o_v, b2_v, acc):
    s = lax.axis_index("s")

    def _init(i, _):
        o_v[pl.ds(16 * i, 16)] = jnp.zeros((16,), jnp.float32)
        return 0
    lax.fori_loop(0, ROWS_T // 16, _init, 0)
    pltpu.sync_copy(o_v, acc.at[pl.ds(ROWS_T * s, ROWS_T)])
    plsc.subcore_barrier()

    pltpu.sync_copy(src_hbm.at[pl.ds(s * NCH1, NCH1)], src_v)
    pltpu.sync_copy(dst_hbm.at[pl.ds(s * NCH1, NCH1)], dst_v)
    pltpu.sync_copy(w_hbm.at[pl.ds(s * EP_W1, EP_W1)], w_v)
    pltpu.sync_copy(h2s_hbm, h2s_v)

    def _chunk(j, _):
        for m in range(CHUNK // 16):
            idx = src_v[j, pl.ds(16 * m, 16)]
            hv = plsc.load_gather(h2s_v, [idx])
            wv = w_v[pl.ds(j * CHUNK + 16 * m, 16)]
            msg_v[pl.ds(16 * m, 16)] = hv * wv
        pltpu.sync_copy(msg_v, acc.at[dst_v.at[j]], add=True)
        return 0
    lax.fori_loop(0, NCH1, _chunk, 0)
    plsc.subcore_barrier()

    pltpu.sync_copy(acc.at[pl.ds(ROWS_T * s, ROWS_T)], a_v)
    pltpu.sync_copy(dis_hbm.at[pl.ds(ROWS_T * s, ROWS_T)], dis_v)
    pltpu.sync_copy(b2_hbm, b2_v)
    bv = b2_v[...]

    def _node(i, _):
        sl = pl.ds(16 * i, 16)
        hv = h2s_v[pl.ds(ROWS_T * s + 16 * i, 16)]
        z = dis_v[sl] * (a_v[sl] + hv) + bv
        o_v[sl] = 1.0 / (1.0 + jnp.exp(-z))
        return 0
    lax.fori_loop(0, ROWS_T // 16, _node, 0)
    pltpu.sync_copy(o_v, out_hbm.at[pl.ds(ROWS_T * s, ROWS_T)])


def kernel(x, edge_index, edge_attr, W1, b1, W2, b2):
    x_p = jnp.pad(x, ((0, NP - N), (0, 0)))
    src_f = jnp.pad(edge_index[0], (0, EP - E))
    dst_f = jnp.pad(edge_index[1], (0, EP - E))
    src = src_f.reshape(NW * NCH, CHUNK)
    dst = dst_f.reshape(NW * NCH, CHUNK)
    w_p = jnp.pad(edge_attr, (0, EP - E))

    dp = _sc_deg(dst, w_p)
    h1s, dis2 = _tc_b(x_p, W1, dp[0][:, None], dp[1][:, None])
    p = _sc_rows(src_f, dst_f.reshape(NW * NCH_C, CH_C), w_p, h1s)
    emb_p, h2s2 = _tc_d(p[0], p[1], h1s, dis2, b1[None, :], W2)
    b2v = jnp.broadcast_to(b2, (16,)).astype(jnp.float32)
    out_flat = _sc_l2(src, dst, w_p, h2s2[:, 0], dis2[:, 0], b2v)
    return (out_flat[:N, None], emb_p[:N])


# speedup vs baseline: 26.3413x; 2.0921x over previous
import functools

import jax
import jax.numpy as jnp
from jax import lax
from jax.experimental import pallas as pl
from jax.experimental.pallas import tpu as pltpu
from jax.experimental.pallas import tpu_sc as plsc

N = 10000
D = 128
E = 320000

NP = 10240
NW = 32
CHUNK = 128
EP_W = 10240
NCH = EP_W // CHUNK
EP = EP_W * NW
ROWS_T = NP // 16

_mesh2 = plsc.VectorSubcoreMesh(core_axis_name="c", subcore_axis_name="s",
                                num_cores=2)
_mesh1 = plsc.VectorSubcoreMesh(core_axis_name="c", subcore_axis_name="s",
                                num_cores=1)


@functools.partial(
    pl.kernel,
    mesh=_mesh2,
    compiler_params=pltpu.CompilerParams(needs_layout_passes=False),
    out_type=jax.ShapeDtypeStruct((2, NP), jnp.float32),
    scratch_types=[
        pltpu.VMEM((NCH, CHUNK), jnp.int32),
        pltpu.VMEM((EP_W,), jnp.float32),
        pltpu.VMEM((ROWS_T,), jnp.float32),
        pltpu.VMEM_SHARED((NP,), jnp.float32),
    ],
)
def _sc_deg(dst_hbm, w_hbm, dp_hbm, dst_v, w_v, stage_v, acc):
    c = lax.axis_index("c")
    s = lax.axis_index("s")
    wid = c * 16 + s

    def _init(i, _):
        stage_v[pl.ds(16 * i, 16)] = jnp.full((16,), 0.5, jnp.float32)
        return 0
    lax.fori_loop(0, ROWS_T // 16, _init, 0)
    pltpu.sync_copy(stage_v, acc.at[pl.ds(ROWS_T * s, ROWS_T)])
    plsc.subcore_barrier()

    pltpu.sync_copy(dst_hbm.at[pl.ds(wid * NCH, NCH)], dst_v)
    pltpu.sync_copy(w_hbm.at[pl.ds(wid * EP_W, EP_W)], w_v)

    def _chunk(j, _):
        pltpu.sync_copy(w_v.at[pl.ds(j * CHUNK, CHUNK)],
                        acc.at[dst_v.at[j]], add=True)
        return 0
    lax.fori_loop(0, NCH, _chunk, 0)
    plsc.subcore_barrier()

    pltpu.sync_copy(acc.at[pl.ds(ROWS_T * s, ROWS_T)],
                    dp_hbm.at[c, pl.ds(ROWS_T * s, ROWS_T)])


def _tc_b_body(x_ref, w1_ref, d0_ref, d1_ref, h1s_ref, dis_ref):
    deg = d0_ref[...] + d1_ref[...]
    dis = lax.rsqrt(deg)
    h1s_ref[...] = jnp.dot(x_ref[...], w1_ref[...],
                           preferred_element_type=jnp.float32) * dis
    dis_ref[...] = dis


def _tc_b(x_p, W1, d0, d1):
    grid = (NP // 128,)
    return pl.pallas_call(
        _tc_b_body,
        grid=grid,
        in_specs=[
            pl.BlockSpec((128, D), lambda i: (i, 0)),
            pl.BlockSpec((D, D), lambda i: (0, 0)),
            pl.BlockSpec((128, 1), lambda i: (i, 0)),
            pl.BlockSpec((128, 1), lambda i: (i, 0)),
        ],
        out_specs=[
            pl.BlockSpec((128, D), lambda i: (i, 0)),
            pl.BlockSpec((128, 1), lambda i: (i, 0)),
        ],
        out_shape=[
            jax.ShapeDtypeStruct((NP, D), jnp.float32),
            jax.ShapeDtypeStruct((NP, 1), jnp.float32),
        ],
    )(x_p, W1, d0, d1)


CH_C = 64
NCH_C = EP_W // CH_C


@functools.partial(
    pl.kernel,
    mesh=_mesh2,
    compiler_params=pltpu.CompilerParams(needs_layout_passes=False),
    out_type=jax.ShapeDtypeStruct((2, NP, D), jnp.float32),
    scratch_types=[
        pltpu.VMEM((EP_W,), jnp.int32),
        pltpu.VMEM((NCH_C, CH_C), jnp.int32),
        pltpu.VMEM((CH_C,), jnp.float32),
        pltpu.VMEM((CH_C,), jnp.float32),
        pltpu.VMEM((CH_C, D), jnp.float32),
        pltpu.VMEM((CH_C, D), jnp.float32),
        pltpu.SemaphoreType.DMA,
        pltpu.SemaphoreType.DMA,
        pltpu.SemaphoreType.DMA,
        pltpu.SemaphoreType.DMA,
        pltpu.SemaphoreType.DMA,
        pltpu.SemaphoreType.DMA,
        pltpu.VMEM_SHARED((NP, D), jnp.float32),
    ],
)
def _sc_rows(src_hbm, dst_hbm, w_hbm, h1s_hbm, p_hbm,
             src_v, dst_v, wb0, wb1, rows0, rows1,
             g0, g1, gw0, gw1, s0, s1, acc):
    c = lax.axis_index("c")
    s = lax.axis_index("s")
    wid = c * 16 + s

    def _zrow(i, _):
        for k in range(D // 16):
            rows0[i, pl.ds(16 * k, 16)] = jnp.zeros((16,), jnp.float32)
        return 0
    lax.fori_loop(0, CH_C, _zrow, 0)
    for m in range(ROWS_T // CH_C):
        pltpu.sync_copy(rows0, acc.at[pl.ds(ROWS_T * s + CH_C * m, CH_C)])
    plsc.subcore_barrier()

    pltpu.sync_copy(src_hbm.at[pl.ds(wid * EP_W, EP_W)], src_v)
    pltpu.sync_copy(dst_hbm.at[pl.ds(wid * NCH_C, NCH_C)], dst_v)

    def _issue(n, rowsb, wbuf, gs, gws):
        pltpu.async_copy(h1s_hbm.at[src_v.at[pl.ds(n * CH_C, CH_C)]],
                         rowsb, gs)
        pltpu.async_copy(w_hbm.at[pl.ds(wid * EP_W + n * CH_C, CH_C)],
                         wbuf, gws)

    def _wait(n, rowsb, wbuf, gs, gws):
        pltpu.make_async_copy(h1s_hbm.at[src_v.at[pl.ds(n * CH_C, CH_C)]],
                              rowsb, gs).wait()
        pltpu.make_async_copy(w_hbm.at[pl.ds(wid * EP_W + n * CH_C, CH_C)],
                              wbuf, gws).wait()

    def _scale(buf, wbuf):
        def _edge(e, _):
            wv = plsc.load_gather(wbuf, [jnp.full((16,), e, jnp.int32)])
            for k in range(D // 16):
                buf[e, pl.ds(16 * k, 16)] = buf[e, pl.ds(16 * k, 16)] * wv
            return 0
        lax.fori_loop(0, CH_C, _edge, 0, unroll=4)

    _issue(0, rows0, wb0, g0, gw0)
    _issue(1, rows1, wb1, g1, gw1)

    def _pair(t, _):
        c0 = 2 * t
        c1 = 2 * t + 1
        _wait(c0, rows0, wb0, g0, gw0)
        _scale(rows0, wb0)
        cp0 = pltpu.async_copy(rows0, acc.at[dst_v.at[c0]], s0, add=True)
        _wait(c1, rows1, wb1, g1, gw1)
        _scale(rows1, wb1)
        cp1 = pltpu.async_copy(rows1, acc.at[dst_v.at[c1]], s1, add=True)
        n0 = jnp.minimum(c0 + 2, NCH_C - 1)
        n1 = jnp.minimum(c1 + 2, NCH_C - 1)
        cp0.wait()
        _issue(n0, rows0, wb0, g0, gw0)
        cp1.wait()
        _issue(n1, rows1, wb1, g1, gw1)
        return 0
    lax.fori_loop(0, NCH_C // 2, _pair, 0)
    _wait(NCH_C - 1, rows0, wb0, g0, gw0)
    _wait(NCH_C - 1, rows1, wb1, g1, gw1)
    plsc.subcore_barrier()

    for m in range(ROWS_T // CH_C):
        r = ROWS_T * s + CH_C * m
        pltpu.sync_copy(acc.at[pl.ds(r, CH_C)], p_hbm.at[c, pl.ds(r, CH_C)])


def _tc_d_body(p0_ref, p1_ref, h1s_ref, dis_ref, b1_ref, w2_ref,
               emb_ref, h2s_ref):
    dis = dis_ref[...]
    s = dis * (p0_ref[...] + p1_ref[...] + h1s_ref[...]) + b1_ref[...]
    h = jnp.where(s > 0, s, jnp.exp(s) - 1.0)
    emb_ref[...] = h
    h2s_ref[...] = jnp.dot(h * dis, w2_ref[...],
                           preferred_element_type=jnp.float32)


def _tc_d(p0, p1, h1s, dis2, b1r, W2):
    grid = (NP // 128,)
    return pl.pallas_call(
        _tc_d_body,
        grid=grid,
        in_specs=[
            pl.BlockSpec((128, D), lambda i: (i, 0)),
            pl.BlockSpec((128, D), lambda i: (i, 0)),
            pl.BlockSpec((128, D), lambda i: (i, 0)),
            pl.BlockSpec((128, 1), lambda i: (i, 0)),
            pl.BlockSpec((1, D), lambda i: (0, 0)),
            pl.BlockSpec((D, 1), lambda i: (0, 0)),
        ],
        out_specs=[
            pl.BlockSpec((128, D), lambda i: (i, 0)),
            pl.BlockSpec((128, 1), lambda i: (i, 0)),
        ],
        out_shape=[
            jax.ShapeDtypeStruct((NP, D), jnp.float32),
            jax.ShapeDtypeStruct((NP, 1), jnp.float32),
        ],
    )(p0, p1, h1s, dis2, b1r, W2)


EP_W1 = EP // 16
NCH1 = EP_W1 // CHUNK


@functools.partial(
    pl.kernel,
    mesh=_mesh1,
    compiler_params=pltpu.CompilerParams(needs_layout_passes=False),
    out_type=jax.ShapeDtypeStruct((NP,), jnp.float32),
    scratch_types=[
        pltpu.VMEM((NCH1, CHUNK), jnp.int32),
        pltpu.VMEM((NCH1, CHUNK), jnp.int32),
        pltpu.VMEM((EP_W1,), jnp.float32),
        pltpu.VMEM((NP,), jnp.float32),
        pltpu.VMEM((CHUNK,), jnp.float32),
        pltpu.VMEM((ROWS_T,), jnp.float32),
        pltpu.VMEM((ROWS_T,), jnp.float32),
        pltpu.VMEM((ROWS_T,), jnp.float32),
        pltpu.VMEM((16,), jnp.float32),
        pltpu.VMEM_SHARED((NP,), jnp.float32),
    ],
)
def _sc_l2(src_hbm, dst_hbm, w_hbm, h2s_hbm, dis_hbm, b2_hbm, out_hbm,
           src_v, dst_v, w_v, h2s_v, msg_v, a_v, dis_v, o_v, b2_v, acc):
    s = lax.axis_index("s")

    def _init(i, _):
        o_v[pl.ds(16 * i, 16)] = jnp.zeros((16,), jnp.float32)
        return 0
    lax.fori_loop(0, ROWS_T // 16, _init, 0)
    pltpu.sync_copy(o_v, acc.at[pl.ds(ROWS_T * s, ROWS_T)])
    plsc.subcore_barrier()

    pltpu.sync_copy(src_hbm.at[pl.ds(s * NCH1, NCH1)], src_v)
    pltpu.sync_copy(dst_hbm.at[pl.ds(s * NCH1, NCH1)], dst_v)
    pltpu.sync_copy(w_hbm.at[pl.ds(s * EP_W1, EP_W1)], w_v)
    pltpu.sync_copy(h2s_hbm, h2s_v)

    def _chunk(j, _):
        for m in range(CHUNK // 16):
            idx = src_v[j, pl.ds(16 * m, 16)]
            hv = plsc.load_gather(h2s_v, [idx])
            wv = w_v[pl.ds(j * CHUNK + 16 * m, 16)]
            msg_v[pl.ds(16 * m, 16)] = hv * wv
        pltpu.sync_copy(msg_v, acc.at[dst_v.at[j]], add=True)
        return 0
    lax.fori_loop(0, NCH1, _chunk, 0)
    plsc.subcore_barrier()

    pltpu.sync_copy(acc.at[pl.ds(ROWS_T * s, ROWS_T)], a_v)
    pltpu.sync_copy(dis_hbm.at[pl.ds(ROWS_T * s, ROWS_T)], dis_v)
    pltpu.sync_copy(b2_hbm, b2_v)
    bv = b2_v[...]

    def _node(i, _):
        sl = pl.ds(16 * i, 16)
        hv = h2s_v[pl.ds(ROWS_T * s + 16 * i, 16)]
        z = dis_v[sl] * (a_v[sl] + hv) + bv
        o_v[sl] = 1.0 / (1.0 + jnp.exp(-z))
        return 0
    lax.fori_loop(0, ROWS_T // 16, _node, 0)
    pltpu.sync_copy(o_v, out_hbm.at[pl.ds(ROWS_T * s, ROWS_T)])


def kernel(x, edge_index, edge_attr, W1, b1, W2, b2):
    x_p = jnp.pad(x, ((0, NP - N), (0, 0)))
    pad_idx = jnp.arange(EP - E, dtype=jnp.int32) % N
    src_f = jnp.concatenate([edge_index[0], pad_idx])
    dst_f = jnp.concatenate([edge_index[1], pad_idx])
    src = src_f.reshape(NW * NCH, CHUNK)
    dst = dst_f.reshape(NW * NCH, CHUNK)
    w_p = jnp.pad(edge_attr, (0, EP - E))

    dp = _sc_deg(dst, w_p)
    h1s, dis2 = _tc_b(x_p, W1, dp[0][:, None], dp[1][:, None])
    p = _sc_rows(src_f, dst_f.reshape(NW * NCH_C, CH_C), w_p, h1s)
    emb_p, h2s2 = _tc_d(p[0], p[1], h1s, dis2, b1[None, :], W2)
    b2v = jnp.broadcast_to(b2, (16,)).astype(jnp.float32)
    out_flat = _sc_l2(src, dst, w_p, h2s2[:, 0], dis2[:, 0], b2v)
    return (out_flat[:N, None], emb_p[:N])


# speedup vs baseline: 32.5168x; 1.2344x over previous
import functools

import jax
import jax.numpy as jnp
from jax import lax
from jax.experimental import pallas as pl
from jax.experimental.pallas import tpu as pltpu
from jax.experimental.pallas import tpu_sc as plsc

N = 10000
D = 128
E = 320000

NP = 10240
NW = 32
CHUNK = 128
EP_W = 10240
NCH = EP_W // CHUNK
EP = EP_W * NW
ROWS_T = NP // 16

_mesh2 = plsc.VectorSubcoreMesh(core_axis_name="c", subcore_axis_name="s",
                                num_cores=2)
_mesh1 = plsc.VectorSubcoreMesh(core_axis_name="c", subcore_axis_name="s",
                                num_cores=1)


@functools.partial(
    pl.kernel,
    mesh=_mesh2,
    compiler_params=pltpu.CompilerParams(needs_layout_passes=False),
    out_type=jax.ShapeDtypeStruct((2, NP), jnp.float32),
    scratch_types=[
        pltpu.VMEM((NCH, CHUNK), jnp.int32),
        pltpu.VMEM((EP_W,), jnp.float32),
        pltpu.VMEM((ROWS_T,), jnp.float32),
        pltpu.VMEM_SHARED((NP,), jnp.float32),
    ],
)
def _sc_deg(dst_hbm, w_hbm, dp_hbm, dst_v, w_v, stage_v, acc):
    c = lax.axis_index("c")
    s = lax.axis_index("s")
    wid = c * 16 + s

    def _init(i, _):
        stage_v[pl.ds(16 * i, 16)] = jnp.full((16,), 0.5, jnp.float32)
        return 0
    lax.fori_loop(0, ROWS_T // 16, _init, 0)
    pltpu.sync_copy(stage_v, acc.at[pl.ds(ROWS_T * s, ROWS_T)])
    plsc.subcore_barrier()

    pltpu.sync_copy(dst_hbm.at[pl.ds(wid * NCH, NCH)], dst_v)
    pltpu.sync_copy(w_hbm.at[pl.ds(wid * EP_W, EP_W)], w_v)

    def _chunk(j, _):
        pltpu.sync_copy(w_v.at[pl.ds(j * CHUNK, CHUNK)],
                        acc.at[dst_v.at[j]], add=True)
        return 0
    lax.fori_loop(0, NCH, _chunk, 0)
    plsc.subcore_barrier()

    pltpu.sync_copy(acc.at[pl.ds(ROWS_T * s, ROWS_T)],
                    dp_hbm.at[c, pl.ds(ROWS_T * s, ROWS_T)])


def _tc_b_body(x_ref, w1_ref, d0_ref, d1_ref, h1s_ref, dis_ref):
    deg = d0_ref[...] + d1_ref[...]
    dis = lax.rsqrt(deg)
    h1s_ref[...] = jnp.dot(x_ref[...], w1_ref[...],
                           preferred_element_type=jnp.float32) * dis
    dis_ref[...] = dis


RB = 1280


def _tc_b(x_p, W1, d0, d1):
    grid = (NP // RB,)
    return pl.pallas_call(
        _tc_b_body,
        grid=grid,
        in_specs=[
            pl.BlockSpec((RB, D), lambda i: (i, 0)),
            pl.BlockSpec((D, D), lambda i: (0, 0)),
            pl.BlockSpec((RB, 1), lambda i: (i, 0)),
            pl.BlockSpec((RB, 1), lambda i: (i, 0)),
        ],
        out_specs=[
            pl.BlockSpec((RB, D), lambda i: (i, 0)),
            pl.BlockSpec((RB, 1), lambda i: (i, 0)),
        ],
        out_shape=[
            jax.ShapeDtypeStruct((NP, D), jnp.float32),
            jax.ShapeDtypeStruct((NP, 1), jnp.float32),
        ],
    )(x_p, W1, d0, d1)


CH_C = 64
NCH_C = EP_W // CH_C


@functools.partial(
    pl.kernel,
    mesh=_mesh2,
    compiler_params=pltpu.CompilerParams(needs_layout_passes=False),
    out_type=jax.ShapeDtypeStruct((2, NP, D), jnp.float32),
    scratch_types=[
        pltpu.VMEM((EP_W,), jnp.int32),
        pltpu.VMEM((NCH_C, CH_C), jnp.int32),
        pltpu.VMEM((CH_C,), jnp.float32),
        pltpu.VMEM((CH_C,), jnp.float32),
        pltpu.VMEM((CH_C, D), jnp.float32),
        pltpu.VMEM((CH_C, D), jnp.float32),
        pltpu.SemaphoreType.DMA,
        pltpu.SemaphoreType.DMA,
        pltpu.SemaphoreType.DMA,
        pltpu.SemaphoreType.DMA,
        pltpu.SemaphoreType.DMA,
        pltpu.SemaphoreType.DMA,
        pltpu.VMEM_SHARED((NP, D), jnp.float32),
    ],
)
def _sc_rows(src_hbm, dst_hbm, w_hbm, h1s_hbm, p_hbm,
             src_v, dst_v, wb0, wb1, rows0, rows1,
             g0, g1, gw0, gw1, s0, s1, acc):
    c = lax.axis_index("c")
    s = lax.axis_index("s")
    wid = c * 16 + s

    def _zrow(i, _):
        for k in range(D // 16):
            rows0[i, pl.ds(16 * k, 16)] = jnp.zeros((16,), jnp.float32)
        return 0
    lax.fori_loop(0, CH_C, _zrow, 0)
    for m in range(ROWS_T // CH_C):
        pltpu.sync_copy(rows0, acc.at[pl.ds(ROWS_T * s + CH_C * m, CH_C)])
    plsc.subcore_barrier()

    pltpu.sync_copy(src_hbm.at[pl.ds(wid * EP_W, EP_W)], src_v)
    pltpu.sync_copy(dst_hbm.at[pl.ds(wid * NCH_C, NCH_C)], dst_v)

    def _issue(n, rowsb, wbuf, gs, gws):
        pltpu.async_copy(h1s_hbm.at[src_v.at[pl.ds(n * CH_C, CH_C)]],
                         rowsb, gs)
        pltpu.async_copy(w_hbm.at[pl.ds(wid * EP_W + n * CH_C, CH_C)],
                         wbuf, gws)

    def _wait(n, rowsb, wbuf, gs, gws):
        pltpu.make_async_copy(h1s_hbm.at[src_v.at[pl.ds(n * CH_C, CH_C)]],
                              rowsb, gs).wait()
        pltpu.make_async_copy(w_hbm.at[pl.ds(wid * EP_W + n * CH_C, CH_C)],
                              wbuf, gws).wait()

    def _scale(buf, wbuf):
        def _edge(e, _):
            wv = plsc.load_gather(wbuf, [jnp.full((16,), e, jnp.int32)])
            for k in range(D // 16):
                buf[e, pl.ds(16 * k, 16)] = buf[e, pl.ds(16 * k, 16)] * wv
            return 0
        lax.fori_loop(0, CH_C, _edge, 0, unroll=4)

    _issue(0, rows0, wb0, g0, gw0)
    _issue(1, rows1, wb1, g1, gw1)

    def _pair(t, _):
        c0 = 2 * t
        c1 = 2 * t + 1
        _wait(c0, rows0, wb0, g0, gw0)
        _scale(rows0, wb0)
        cp0 = pltpu.async_copy(rows0, acc.at[dst_v.at[c0]], s0, add=True)
        _wait(c1, rows1, wb1, g1, gw1)
        _scale(rows1, wb1)
        cp1 = pltpu.async_copy(rows1, acc.at[dst_v.at[c1]], s1, add=True)
        n0 = jnp.minimum(c0 + 2, NCH_C - 1)
        n1 = jnp.minimum(c1 + 2, NCH_C - 1)
        cp0.wait()
        _issue(n0, rows0, wb0, g0, gw0)
        cp1.wait()
        _issue(n1, rows1, wb1, g1, gw1)
        return 0
    lax.fori_loop(0, NCH_C // 2, _pair, 0)
    _wait(NCH_C - 1, rows0, wb0, g0, gw0)
    _wait(NCH_C - 1, rows1, wb1, g1, gw1)
    plsc.subcore_barrier()

    for m in range(ROWS_T // CH_C):
        r = ROWS_T * s + CH_C * m
        pltpu.sync_copy(acc.at[pl.ds(r, CH_C)], p_hbm.at[c, pl.ds(r, CH_C)])


def _tc_d_body(p0_ref, p1_ref, h1s_ref, dis_ref, b1_ref, w2_ref,
               emb_ref, h2s_ref):
    dis = dis_ref[...]
    s = dis * (p0_ref[...] + p1_ref[...] + h1s_ref[...]) + b1_ref[...]
    h = jnp.where(s > 0, s, jnp.exp(s) - 1.0)
    emb_ref[...] = h
    h2s_ref[...] = jnp.dot(h * dis, w2_ref[...],
                           preferred_element_type=jnp.float32)


def _tc_d(p0, p1, h1s, dis2, b1r, W2):
    grid = (NP // RB,)
    return pl.pallas_call(
        _tc_d_body,
        grid=grid,
        in_specs=[
            pl.BlockSpec((RB, D), lambda i: (i, 0)),
            pl.BlockSpec((RB, D), lambda i: (i, 0)),
            pl.BlockSpec((RB, D), lambda i: (i, 0)),
            pl.BlockSpec((RB, 1), lambda i: (i, 0)),
            pl.BlockSpec((1, D), lambda i: (0, 0)),
            pl.BlockSpec((D, 1), lambda i: (0, 0)),
        ],
        out_specs=[
            pl.BlockSpec((RB, D), lambda i: (i, 0)),
            pl.BlockSpec((RB, 1), lambda i: (i, 0)),
        ],
        out_shape=[
            jax.ShapeDtypeStruct((NP, D), jnp.float32),
            jax.ShapeDtypeStruct((NP, 1), jnp.float32),
        ],
    )(p0, p1, h1s, dis2, b1r, W2)


EP_W1 = EP // 16
NCH1 = EP_W1 // CHUNK


@functools.partial(
    pl.kernel,
    mesh=_mesh1,
    compiler_params=pltpu.CompilerParams(needs_layout_passes=False),
    out_type=jax.ShapeDtypeStruct((NP,), jnp.float32),
    scratch_types=[
        pltpu.VMEM((NCH1, CHUNK), jnp.int32),
        pltpu.VMEM((NCH1, CHUNK), jnp.int32),
        pltpu.VMEM((EP_W1,), jnp.float32),
        pltpu.VMEM((NP,), jnp.float32),
        pltpu.VMEM((CHUNK,), jnp.float32),
        pltpu.VMEM((ROWS_T,), jnp.float32),
        pltpu.VMEM((ROWS_T,), jnp.float32),
        pltpu.VMEM((ROWS_T,), jnp.float32),
        pltpu.VMEM((16,), jnp.float32),
        pltpu.VMEM_SHARED((NP,), jnp.float32),
    ],
)
def _sc_l2(src_hbm, dst_hbm, w_hbm, h2s_hbm, dis_hbm, b2_hbm, out_hbm,
           src_v, dst_v, w_v, h2s_v, msg_v, a_v, dis_v, o_v, b2_v, acc):
    s = lax.axis_index("s")

    def _init(i, _):
        o_v[pl.ds(16 * i, 16)] = jnp.zeros((16,), jnp.float32)
        return 0
    lax.fori_loop(0, ROWS_T // 16, _init, 0)
    pltpu.sync_copy(o_v, acc.at[pl.ds(ROWS_T * s, ROWS_T)])
    plsc.subcore_barrier()

    pltpu.sync_copy(src_hbm.at[pl.ds(s * NCH1, NCH1)], src_v)
    pltpu.sync_copy(dst_hbm.at[pl.ds(s * NCH1, NCH1)], dst_v)
    pltpu.sync_copy(w_hbm.at[pl.ds(s * EP_W1, EP_W1)], w_v)
    pltpu.sync_copy(h2s_hbm, h2s_v)

    def _chunk(j, _):
        for m in range(CHUNK // 16):
            idx = src_v[j, pl.ds(16 * m, 16)]
            hv = plsc.load_gather(h2s_v, [idx])
            wv = w_v[pl.ds(j * CHUNK + 16 * m, 16)]
            msg_v[pl.ds(16 * m, 16)] = hv * wv
        pltpu.sync_copy(msg_v, acc.at[dst_v.at[j]], add=True)
        return 0
    lax.fori_loop(0, NCH1, _chunk, 0)
    plsc.subcore_barrier()

    pltpu.sync_copy(acc.at[pl.ds(ROWS_T * s, ROWS_T)], a_v)
    pltpu.sync_copy(dis_hbm.at[pl.ds(ROWS_T * s, ROWS_T)], dis_v)
    pltpu.sync_copy(b2_hbm, b2_v)
    bv = b2_v[...]

    def _node(i, _):
        sl = pl.ds(16 * i, 16)
        hv = h2s_v[pl.ds(ROWS_T * s + 16 * i, 16)]
        z = dis_v[sl] * (a_v[sl] + hv) + bv
        o_v[sl] = 1.0 / (1.0 + jnp.exp(-z))
        return 0
    lax.fori_loop(0, ROWS_T // 16, _node, 0)
    pltpu.sync_copy(o_v, out_hbm.at[pl.ds(ROWS_T * s, ROWS_T)])


def kernel(x, edge_index, edge_attr, W1, b1, W2, b2):
    x_p = jnp.pad(x, ((0, NP - N), (0, 0)))
    pad_idx = jnp.arange(EP - E, dtype=jnp.int32) % N
    src_f = jnp.concatenate([edge_index[0], pad_idx])
    dst_f = jnp.concatenate([edge_index[1], pad_idx])
    src = src_f.reshape(NW * NCH, CHUNK)
    dst = dst_f.reshape(NW * NCH, CHUNK)
    w_p = jnp.pad(edge_attr, (0, EP - E))

    dp = _sc_deg(dst, w_p)
    h1s, dis2 = _tc_b(x_p, W1, dp[0][:, None], dp[1][:, None])
    p = _sc_rows(src_f, dst_f.reshape(NW * NCH_C, CH_C), w_p, h1s)
    emb_p, h2s2 = _tc_d(p[0], p[1], h1s, dis2, b1[None, :], W2)
    b2v = jnp.broadcast_to(b2, (16,)).astype(jnp.float32)
    out_flat = _sc_l2(src, dst, w_p, h2s2[:, 0], dis2[:, 0], b2v)
    return (out_flat[:N, None], emb_p[:N])


# speedup vs baseline: 34.7485x; 1.0686x over previous
import functools

import jax
import jax.numpy as jnp
from jax import lax
from jax.experimental import pallas as pl
from jax.experimental.pallas import tpu as pltpu
from jax.experimental.pallas import tpu_sc as plsc

N = 10000
D = 128
E = 320000

NP = 10240
NW = 32
CHUNK = 128
EP_W = 10240
NCH = EP_W // CHUNK
EP = EP_W * NW
ROWS_T = NP // 16

_mesh2 = plsc.VectorSubcoreMesh(core_axis_name="c", subcore_axis_name="s",
                                num_cores=2)
_mesh1 = plsc.VectorSubcoreMesh(core_axis_name="c", subcore_axis_name="s",
                                num_cores=1)


@functools.partial(
    pl.kernel,
    mesh=_mesh2,
    compiler_params=pltpu.CompilerParams(needs_layout_passes=False),
    out_type=jax.ShapeDtypeStruct((2, NP), jnp.float32),
    scratch_types=[
        pltpu.VMEM((NCH, CHUNK), jnp.int32),
        pltpu.VMEM((EP_W,), jnp.float32),
        pltpu.VMEM((ROWS_T,), jnp.float32),
        pltpu.VMEM_SHARED((NP,), jnp.float32),
    ],
)
def _sc_deg(dst_hbm, w_hbm, dp_hbm, dst_v, w_v, stage_v, acc):
    c = lax.axis_index("c")
    s = lax.axis_index("s")
    wid = c * 16 + s

    def _init(i, _):
        stage_v[pl.ds(16 * i, 16)] = jnp.full((16,), 0.5, jnp.float32)
        return 0
    lax.fori_loop(0, ROWS_T // 16, _init, 0)
    pltpu.sync_copy(stage_v, acc.at[pl.ds(ROWS_T * s, ROWS_T)])
    plsc.subcore_barrier()

    pltpu.sync_copy(dst_hbm.at[pl.ds(wid * NCH, NCH)], dst_v)
    pltpu.sync_copy(w_hbm.at[pl.ds(wid * EP_W, EP_W)], w_v)

    def _chunk(j, _):
        pltpu.sync_copy(w_v.at[pl.ds(j * CHUNK, CHUNK)],
                        acc.at[dst_v.at[j]], add=True)
        return 0
    lax.fori_loop(0, NCH, _chunk, 0)
    plsc.subcore_barrier()

    pltpu.sync_copy(acc.at[pl.ds(ROWS_T * s, ROWS_T)],
                    dp_hbm.at[c, pl.ds(ROWS_T * s, ROWS_T)])


def _tc_b_body(x_ref, w1_ref, d0_ref, d1_ref, h1s_ref, dis_ref):
    deg = d0_ref[...] + d1_ref[...]
    dis = lax.rsqrt(deg)
    h1s_ref[...] = jnp.dot(x_ref[...], w1_ref[...],
                           preferred_element_type=jnp.float32) * dis
    dis_ref[...] = dis


RB = 1280


def _tc_b(x_p, W1, d0, d1):
    grid = (NP // RB,)
    return pl.pallas_call(
        _tc_b_body,
        grid=grid,
        in_specs=[
            pl.BlockSpec((RB, D), lambda i: (i, 0)),
            pl.BlockSpec((D, D), lambda i: (0, 0)),
            pl.BlockSpec((RB, 1), lambda i: (i, 0)),
            pl.BlockSpec((RB, 1), lambda i: (i, 0)),
        ],
        out_specs=[
            pl.BlockSpec((RB, D), lambda i: (i, 0)),
            pl.BlockSpec((RB, 1), lambda i: (i, 0)),
        ],
        out_shape=[
            jax.ShapeDtypeStruct((NP, D), jnp.float32),
            jax.ShapeDtypeStruct((NP, 1), jnp.float32),
        ],
    )(x_p, W1, d0, d1)


CH_C = 64
NCH_C = EP_W // CH_C


@functools.partial(
    pl.kernel,
    mesh=_mesh2,
    compiler_params=pltpu.CompilerParams(needs_layout_passes=False),
    out_type=jax.ShapeDtypeStruct((2, NP, D), jnp.float32),
    scratch_types=[
        pltpu.VMEM((EP_W,), jnp.int32),
        pltpu.VMEM((NCH_C, CH_C), jnp.int32),
        pltpu.VMEM((CH_C,), jnp.float32),
        pltpu.VMEM((CH_C,), jnp.float32),
        pltpu.VMEM((CH_C, D), jnp.float32),
        pltpu.VMEM((CH_C, D), jnp.float32),
        pltpu.SemaphoreType.DMA,
        pltpu.SemaphoreType.DMA,
        pltpu.SemaphoreType.DMA,
        pltpu.SemaphoreType.DMA,
        pltpu.SemaphoreType.DMA,
        pltpu.SemaphoreType.DMA,
        pltpu.VMEM_SHARED((NP, D), jnp.float32),
    ],
)
def _sc_rows(src_hbm, dst_hbm, w_hbm, h1s_hbm, p_hbm,
             src_v, dst_v, wb0, wb1, rows0, rows1,
             g0, g1, gw0, gw1, s0, s1, acc):
    c = lax.axis_index("c")
    s = lax.axis_index("s")
    wid = c * 16 + s

    def _zrow(i, _):
        for k in range(D // 16):
            rows0[i, pl.ds(16 * k, 16)] = jnp.zeros((16,), jnp.float32)
        return 0
    lax.fori_loop(0, CH_C, _zrow, 0)
    for m in range(ROWS_T // CH_C):
        pltpu.sync_copy(rows0, acc.at[pl.ds(ROWS_T * s + CH_C * m, CH_C)])
    plsc.subcore_barrier()

    pltpu.sync_copy(src_hbm.at[pl.ds(wid * EP_W, EP_W)], src_v)
    pltpu.sync_copy(dst_hbm.at[pl.ds(wid * NCH_C, NCH_C)], dst_v)

    def _issue(n, rowsb, wbuf, gs, gws):
        pltpu.async_copy(h1s_hbm.at[src_v.at[pl.ds(n * CH_C, CH_C)]],
                         rowsb, gs)
        pltpu.async_copy(w_hbm.at[pl.ds(wid * EP_W + n * CH_C, CH_C)],
                         wbuf, gws)

    def _wait(n, rowsb, wbuf, gs, gws):
        pltpu.make_async_copy(h1s_hbm.at[src_v.at[pl.ds(n * CH_C, CH_C)]],
                              rowsb, gs).wait()
        pltpu.make_async_copy(w_hbm.at[pl.ds(wid * EP_W + n * CH_C, CH_C)],
                              wbuf, gws).wait()

    def _scale(buf, wbuf):
        for m in range(CH_C // 16):
            w16 = wbuf[pl.ds(16 * m, 16)]
            for t in range(16):
                wv = jnp.take_along_axis(w16, jnp.full((16,), t, jnp.int32),
                                         axis=0)
                e = 16 * m + t
                for k in range(D // 16):
                    buf[e, pl.ds(16 * k, 16)] = buf[e, pl.ds(16 * k, 16)] * wv

    _issue(0, rows0, wb0, g0, gw0)
    _issue(1, rows1, wb1, g1, gw1)

    def _pair(t, _):
        c0 = 2 * t
        c1 = 2 * t + 1
        _wait(c0, rows0, wb0, g0, gw0)
        _scale(rows0, wb0)
        cp0 = pltpu.async_copy(rows0, acc.at[dst_v.at[c0]], s0, add=True)
        _wait(c1, rows1, wb1, g1, gw1)
        _scale(rows1, wb1)
        cp1 = pltpu.async_copy(rows1, acc.at[dst_v.at[c1]], s1, add=True)
        n0 = jnp.minimum(c0 + 2, NCH_C - 1)
        n1 = jnp.minimum(c1 + 2, NCH_C - 1)
        cp0.wait()
        _issue(n0, rows0, wb0, g0, gw0)
        cp1.wait()
        _issue(n1, rows1, wb1, g1, gw1)
        return 0
    lax.fori_loop(0, NCH_C // 2, _pair, 0)
    _wait(NCH_C - 1, rows0, wb0, g0, gw0)
    _wait(NCH_C - 1, rows1, wb1, g1, gw1)
    plsc.subcore_barrier()

    for m in range(ROWS_T // CH_C):
        r = ROWS_T * s + CH_C * m
        pltpu.sync_copy(acc.at[pl.ds(r, CH_C)], p_hbm.at[c, pl.ds(r, CH_C)])


def _tc_d_body(p0_ref, p1_ref, h1s_ref, dis_ref, b1_ref, w2_ref,
               emb_ref, h2s_ref):
    dis = dis_ref[...]
    s = dis * (p0_ref[...] + p1_ref[...] + h1s_ref[...]) + b1_ref[...]
    h = jnp.where(s > 0, s, jnp.exp(s) - 1.0)
    emb_ref[...] = h
    h2s_ref[...] = jnp.dot(h * dis, w2_ref[...],
                           preferred_element_type=jnp.float32)


def _tc_d(p0, p1, h1s, dis2, b1r, W2):
    grid = (NP // RB,)
    return pl.pallas_call(
        _tc_d_body,
        grid=grid,
        in_specs=[
            pl.BlockSpec((RB, D), lambda i: (i, 0)),
            pl.BlockSpec((RB, D), lambda i: (i, 0)),
            pl.BlockSpec((RB, D), lambda i: (i, 0)),
            pl.BlockSpec((RB, 1), lambda i: (i, 0)),
            pl.BlockSpec((1, D), lambda i: (0, 0)),
            pl.BlockSpec((D, 1), lambda i: (0, 0)),
        ],
        out_specs=[
            pl.BlockSpec((RB, D), lambda i: (i, 0)),
            pl.BlockSpec((RB, 1), lambda i: (i, 0)),
        ],
        out_shape=[
            jax.ShapeDtypeStruct((NP, D), jnp.float32),
            jax.ShapeDtypeStruct((NP, 1), jnp.float32),
        ],
    )(p0, p1, h1s, dis2, b1r, W2)


EP_W1 = EP // 16
NCH1 = EP_W1 // CHUNK


@functools.partial(
    pl.kernel,
    mesh=_mesh1,
    compiler_params=pltpu.CompilerParams(needs_layout_passes=False),
    out_type=jax.ShapeDtypeStruct((NP,), jnp.float32),
    scratch_types=[
        pltpu.VMEM((NCH1, CHUNK), jnp.int32),
        pltpu.VMEM((NCH1, CHUNK), jnp.int32),
        pltpu.VMEM((EP_W1,), jnp.float32),
        pltpu.VMEM((NP,), jnp.float32),
        pltpu.VMEM((CHUNK,), jnp.float32),
        pltpu.VMEM((ROWS_T,), jnp.float32),
        pltpu.VMEM((ROWS_T,), jnp.float32),
        pltpu.VMEM((ROWS_T,), jnp.float32),
        pltpu.VMEM((16,), jnp.float32),
        pltpu.VMEM_SHARED((NP,), jnp.float32),
    ],
)
def _sc_l2(src_hbm, dst_hbm, w_hbm, h2s_hbm, dis_hbm, b2_hbm, out_hbm,
           src_v, dst_v, w_v, h2s_v, msg_v, a_v, dis_v, o_v, b2_v, acc):
    s = lax.axis_index("s")

    def _init(i, _):
        o_v[pl.ds(16 * i, 16)] = jnp.zeros((16,), jnp.float32)
        return 0
    lax.fori_loop(0, ROWS_T // 16, _init, 0)
    pltpu.sync_copy(o_v, acc.at[pl.ds(ROWS_T * s, ROWS_T)])
    plsc.subcore_barrier()

    pltpu.sync_copy(src_hbm.at[pl.ds(s * NCH1, NCH1)], src_v)
    pltpu.sync_copy(dst_hbm.at[pl.ds(s * NCH1, NCH1)], dst_v)
    pltpu.sync_copy(w_hbm.at[pl.ds(s * EP_W1, EP_W1)], w_v)
    pltpu.sync_copy(h2s_hbm, h2s_v)

    def _chunk(j, _):
        for m in range(CHUNK // 16):
            idx = src_v[j, pl.ds(16 * m, 16)]
            hv = plsc.load_gather(h2s_v, [idx])
            wv = w_v[pl.ds(j * CHUNK + 16 * m, 16)]
            msg_v[pl.ds(16 * m, 16)] = hv * wv
        pltpu.sync_copy(msg_v, acc.at[dst_v.at[j]], add=True)
        return 0
    lax.fori_loop(0, NCH1, _chunk, 0)
    plsc.subcore_barrier()

    pltpu.sync_copy(acc.at[pl.ds(ROWS_T * s, ROWS_T)], a_v)
    pltpu.sync_copy(dis_hbm.at[pl.ds(ROWS_T * s, ROWS_T)], dis_v)
    pltpu.sync_copy(b2_hbm, b2_v)
    bv = b2_v[...]

    def _node(i, _):
        sl = pl.ds(16 * i, 16)
        hv = h2s_v[pl.ds(ROWS_T * s + 16 * i, 16)]
        z = dis_v[sl] * (a_v[sl] + hv) + bv
        o_v[sl] = 1.0 / (1.0 + jnp.exp(-z))
        return 0
    lax.fori_loop(0, ROWS_T // 16, _node, 0)
    pltpu.sync_copy(o_v, out_hbm.at[pl.ds(ROWS_T * s, ROWS_T)])


def kernel(x, edge_index, edge_attr, W1, b1, W2, b2):
    x_p = jnp.pad(x, ((0, NP - N), (0, 0)))
    pad_idx = jnp.arange(EP - E, dtype=jnp.int32) % N
    src_f = jnp.concatenate([edge_index[0], pad_idx])
    dst_f = jnp.concatenate([edge_index[1], pad_idx])
    src = src_f.reshape(NW * NCH, CHUNK)
    dst = dst_f.reshape(NW * NCH, CHUNK)
    w_p = jnp.pad(edge_attr, (0, EP - E))

    dp = _sc_deg(dst, w_p)
    h1s, dis2 = _tc_b(x_p, W1, dp[0][:, None], dp[1][:, None])
    p = _sc_rows(src_f, dst_f.reshape(NW * NCH_C, CH_C), w_p, h1s)
    emb_p, h2s2 = _tc_d(p[0], p[1], h1s, dis2, b1[None, :], W2)
    b2v = jnp.broadcast_to(b2, (16,)).astype(jnp.float32)
    out_flat = _sc_l2(src, dst, w_p, h2s2[:, 0], dis2[:, 0], b2v)
    return (out_flat[:N, None], emb_p[:N])


# speedup vs baseline: 36.2023x; 1.0418x over previous
import functools

import jax
import jax.numpy as jnp
from jax import lax
from jax.experimental import pallas as pl
from jax.experimental.pallas import tpu as pltpu
from jax.experimental.pallas import tpu_sc as plsc

N = 10000
D = 128
E = 320000

NP = 10240
NW = 32
CHUNK = 128
EP_W = 10240
NCH = EP_W // CHUNK
EP = EP_W * NW
ROWS_T = NP // 16

_mesh2 = plsc.VectorSubcoreMesh(core_axis_name="c", subcore_axis_name="s",
                                num_cores=2)
_mesh1 = plsc.VectorSubcoreMesh(core_axis_name="c", subcore_axis_name="s",
                                num_cores=1)


@functools.partial(
    pl.kernel,
    mesh=_mesh2,
    compiler_params=pltpu.CompilerParams(needs_layout_passes=False),
    out_type=[jax.ShapeDtypeStruct((NP,), jnp.float32),
              jax.ShapeDtypeStruct((NP,), jnp.float32)],
    scratch_types=[
        pltpu.VMEM((NCH, CHUNK), jnp.int32),
        pltpu.VMEM((EP_W,), jnp.float32),
        pltpu.VMEM((ROWS_T,), jnp.float32),
        pltpu.SemaphoreType.DMA,
        pltpu.VMEM_SHARED((NP,), jnp.float32),
    ],
)
def _sc_deg(dst_hbm, w_hbm, dp0_hbm, dp1_hbm, dst_v, w_v, stage_v, ssem, acc):
    c = lax.axis_index("c")
    s = lax.axis_index("s")
    wid = c * 16 + s

    def _init(i, _):
        stage_v[pl.ds(16 * i, 16)] = jnp.full((16,), 0.5, jnp.float32)
        return 0
    lax.fori_loop(0, ROWS_T // 16, _init, 0)
    pltpu.sync_copy(stage_v, acc.at[pl.ds(ROWS_T * s, ROWS_T)])
    plsc.subcore_barrier()

    pltpu.sync_copy(dst_hbm.at[pl.ds(wid * NCH, NCH)], dst_v)
    pltpu.sync_copy(w_hbm.at[pl.ds(wid * EP_W, EP_W)], w_v)

    K = 16

    def _chunk(jg, _):
        for u in range(K):
            j = jg * K + u
            pltpu.async_copy(w_v.at[pl.ds(j * CHUNK, CHUNK)],
                             acc.at[dst_v.at[j]], ssem, add=True)
        for u in range(K):
            j = jg * K + u
            pltpu.make_async_copy(w_v.at[pl.ds(j * CHUNK, CHUNK)],
                                  acc.at[dst_v.at[j]], ssem).wait()
        return 0
    lax.fori_loop(0, NCH // K, _chunk, 0)
    plsc.subcore_barrier()

    @pl.when(c == 0)
    def _w0():
        pltpu.sync_copy(acc.at[pl.ds(ROWS_T * s, ROWS_T)],
                        dp0_hbm.at[pl.ds(ROWS_T * s, ROWS_T)])

    @pl.when(c == 1)
    def _w1():
        pltpu.sync_copy(acc.at[pl.ds(ROWS_T * s, ROWS_T)],
                        dp1_hbm.at[pl.ds(ROWS_T * s, ROWS_T)])


def _tc_b_body(x_ref, w1_ref, d0_ref, d1_ref, h1s_ref, dis_ref):
    deg = d0_ref[...] + d1_ref[...]
    dis = lax.rsqrt(deg)
    h1s_ref[...] = jnp.dot(x_ref[...], w1_ref[...],
                           preferred_element_type=jnp.float32) * dis
    dis_ref[...] = dis


RB = 1280


def _tc_b(x_p, W1, d0, d1):
    grid = (NP // RB,)
    return pl.pallas_call(
        _tc_b_body,
        grid=grid,
        in_specs=[
            pl.BlockSpec((RB, D), lambda i: (i, 0)),
            pl.BlockSpec((D, D), lambda i: (0, 0)),
            pl.BlockSpec((RB, 1), lambda i: (i, 0)),
            pl.BlockSpec((RB, 1), lambda i: (i, 0)),
        ],
        out_specs=[
            pl.BlockSpec((RB, D), lambda i: (i, 0)),
            pl.BlockSpec((RB, 1), lambda i: (i, 0)),
        ],
        out_shape=[
            jax.ShapeDtypeStruct((NP, D), jnp.float32),
            jax.ShapeDtypeStruct((NP, 1), jnp.float32),
        ],
    )(x_p, W1, d0, d1)


CH_C = 64
NCH_C = EP_W // CH_C


@functools.partial(
    pl.kernel,
    mesh=_mesh2,
    compiler_params=pltpu.CompilerParams(needs_layout_passes=False),
    out_type=[jax.ShapeDtypeStruct((NP, D), jnp.float32),
              jax.ShapeDtypeStruct((NP, D), jnp.float32)],
    scratch_types=[
        pltpu.VMEM((EP_W,), jnp.int32),
        pltpu.VMEM((NCH_C, CH_C), jnp.int32),
        pltpu.VMEM((CH_C,), jnp.float32),
        pltpu.VMEM((CH_C,), jnp.float32),
        pltpu.VMEM((CH_C, D), jnp.float32),
        pltpu.VMEM((CH_C, D), jnp.float32),
        pltpu.SemaphoreType.DMA,
        pltpu.SemaphoreType.DMA,
        pltpu.SemaphoreType.DMA,
        pltpu.SemaphoreType.DMA,
        pltpu.SemaphoreType.DMA,
        pltpu.SemaphoreType.DMA,
        pltpu.VMEM_SHARED((NP, D), jnp.float32),
    ],
)
def _sc_rows(src_hbm, dst_hbm, w_hbm, h1s_hbm, p0_hbm, p1_hbm,
             src_v, dst_v, wb0, wb1, rows0, rows1,
             g0, g1, gw0, gw1, s0, s1, acc):
    c = lax.axis_index("c")
    s = lax.axis_index("s")
    wid = c * 16 + s

    def _zrow(i, _):
        for k in range(D // 16):
            rows0[i, pl.ds(16 * k, 16)] = jnp.zeros((16,), jnp.float32)
        return 0
    lax.fori_loop(0, CH_C, _zrow, 0)
    for m in range(ROWS_T // CH_C):
        pltpu.sync_copy(rows0, acc.at[pl.ds(ROWS_T * s + CH_C * m, CH_C)])
    plsc.subcore_barrier()

    pltpu.sync_copy(src_hbm.at[pl.ds(wid * EP_W, EP_W)], src_v)
    pltpu.sync_copy(dst_hbm.at[pl.ds(wid * NCH_C, NCH_C)], dst_v)

    def _issue(n, rowsb, wbuf, gs, gws):
        pltpu.async_copy(h1s_hbm.at[src_v.at[pl.ds(n * CH_C, CH_C)]],
                         rowsb, gs)
        pltpu.async_copy(w_hbm.at[pl.ds(wid * EP_W + n * CH_C, CH_C)],
                         wbuf, gws)

    def _wait(n, rowsb, wbuf, gs, gws):
        pltpu.make_async_copy(h1s_hbm.at[src_v.at[pl.ds(n * CH_C, CH_C)]],
                              rowsb, gs).wait()
        pltpu.make_async_copy(w_hbm.at[pl.ds(wid * EP_W + n * CH_C, CH_C)],
                              wbuf, gws).wait()

    def _scale(buf, wbuf):
        for m in range(CH_C // 16):
            w16 = wbuf[pl.ds(16 * m, 16)]
            for t in range(16):
                wv = jnp.take_along_axis(w16, jnp.full((16,), t, jnp.int32),
                                         axis=0)
                e = 16 * m + t
                for k in range(D // 16):
                    buf[e, pl.ds(16 * k, 16)] = buf[e, pl.ds(16 * k, 16)] * wv

    _issue(0, rows0, wb0, g0, gw0)
    _issue(1, rows1, wb1, g1, gw1)

    def _pair(t, _):
        c0 = 2 * t
        c1 = 2 * t + 1
        _wait(c0, rows0, wb0, g0, gw0)
        _scale(rows0, wb0)
        cp0 = pltpu.async_copy(rows0, acc.at[dst_v.at[c0]], s0, add=True)
        _wait(c1, rows1, wb1, g1, gw1)
        _scale(rows1, wb1)
        cp1 = pltpu.async_copy(rows1, acc.at[dst_v.at[c1]], s1, add=True)
        n0 = jnp.minimum(c0 + 2, NCH_C - 1)
        n1 = jnp.minimum(c1 + 2, NCH_C - 1)
        cp0.wait()
        _issue(n0, rows0, wb0, g0, gw0)
        cp1.wait()
        _issue(n1, rows1, wb1, g1, gw1)
        return 0
    lax.fori_loop(0, NCH_C // 2, _pair, 0)
    _wait(NCH_C - 1, rows0, wb0, g0, gw0)
    _wait(NCH_C - 1, rows1, wb1, g1, gw1)
    plsc.subcore_barrier()

    for m in range(ROWS_T // CH_C):
        r = ROWS_T * s + CH_C * m
        @pl.when(c == 0)
        def _w0():
            pltpu.sync_copy(acc.at[pl.ds(r, CH_C)], p0_hbm.at[pl.ds(r, CH_C)])

        @pl.when(c == 1)
        def _w1():
            pltpu.sync_copy(acc.at[pl.ds(r, CH_C)], p1_hbm.at[pl.ds(r, CH_C)])


def _tc_d_body(p0_ref, p1_ref, h1s_ref, dis_ref, b1_ref, w2_ref,
               emb_ref, h2s_ref):
    dis = dis_ref[...]
    s = dis * (p0_ref[...] + p1_ref[...] + h1s_ref[...]) + b1_ref[...]
    h = jnp.where(s > 0, s, jnp.exp(s) - 1.0)
    emb_ref[...] = h
    h2s_ref[...] = jnp.dot(h * dis, w2_ref[...],
                           preferred_element_type=jnp.float32)


def _tc_d(p0, p1, h1s, dis2, b1r, W2):
    grid = (NP // RB,)
    return pl.pallas_call(
        _tc_d_body,
        grid=grid,
        in_specs=[
            pl.BlockSpec((RB, D), lambda i: (i, 0)),
            pl.BlockSpec((RB, D), lambda i: (i, 0)),
            pl.BlockSpec((RB, D), lambda i: (i, 0)),
            pl.BlockSpec((RB, 1), lambda i: (i, 0)),
            pl.BlockSpec((1, D), lambda i: (0, 0)),
            pl.BlockSpec((D, 1), lambda i: (0, 0)),
        ],
        out_specs=[
            pl.BlockSpec((RB, D), lambda i: (i, 0)),
            pl.BlockSpec((RB, 1), lambda i: (i, 0)),
        ],
        out_shape=[
            jax.ShapeDtypeStruct((NP, D), jnp.float32),
            jax.ShapeDtypeStruct((NP, 1), jnp.float32),
        ],
    )(p0, p1, h1s, dis2, b1r, W2)


EP_W1 = EP // 16
NCH1 = EP_W1 // CHUNK


@functools.partial(
    pl.kernel,
    mesh=_mesh1,
    compiler_params=pltpu.CompilerParams(needs_layout_passes=False),
    out_type=jax.ShapeDtypeStruct((NP,), jnp.float32),
    scratch_types=[
        pltpu.VMEM((NCH1, CHUNK), jnp.int32),
        pltpu.VMEM((NCH1, CHUNK), jnp.int32),
        pltpu.VMEM((EP_W1,), jnp.float32),
        pltpu.VMEM((NP,), jnp.float32),
        pltpu.VMEM((CHUNK,), jnp.float32),
        pltpu.VMEM((ROWS_T,), jnp.float32),
        pltpu.VMEM((ROWS_T,), jnp.float32),
        pltpu.VMEM((ROWS_T,), jnp.float32),
        pltpu.VMEM((16,), jnp.float32),
        pltpu.VMEM_SHARED((NP,), jnp.float32),
    ],
)
def _sc_l2(src_hbm, dst_hbm, w_hbm, h2s_hbm, dis_hbm, b2_hbm, out_hbm,
           src_v, dst_v, w_v, h2s_v, msg_v, a_v, dis_v, o_v, b2_v, acc):
    s = lax.axis_index("s")

    def _init(i, _):
        o_v[pl.ds(16 * i, 16)] = jnp.zeros((16,), jnp.float32)
        return 0
    lax.fori_loop(0, ROWS_T // 16, _init, 0)
    pltpu.sync_copy(o_v, acc.at[pl.ds(ROWS_T * s, ROWS_T)])
    plsc.subcore_barrier()

    pltpu.sync_copy(src_hbm.at[pl.ds(s * NCH1, NCH1)], src_v)
    pltpu.sync_copy(dst_hbm.at[pl.ds(s * NCH1, NCH1)], dst_v)
    pltpu.sync_copy(w_hbm.at[pl.ds(s * EP_W1, EP_W1)], w_v)
    pltpu.sync_copy(h2s_hbm, h2s_v)

    def _chunk(j, _):
        for m in range(CHUNK // 16):
            idx = src_v[j, pl.ds(16 * m, 16)]
            hv = plsc.load_gather(h2s_v, [idx])
            wv = w_v[pl.ds(j * CHUNK + 16 * m, 16)]
            msg_v[pl.ds(16 * m, 16)] = hv * wv
        pltpu.sync_copy(msg_v, acc.at[dst_v.at[j]], add=True)
        return 0
    lax.fori_loop(0, NCH1, _chunk, 0)
    plsc.subcore_barrier()

    pltpu.sync_copy(acc.at[pl.ds(ROWS_T * s, ROWS_T)], a_v)
    pltpu.sync_copy(dis_hbm.at[pl.ds(ROWS_T * s, ROWS_T)], dis_v)
    pltpu.sync_copy(b2_hbm, b2_v)
    bv = b2_v[...]

    def _node(i, _):
        sl = pl.ds(16 * i, 16)
        hv = h2s_v[pl.ds(ROWS_T * s + 16 * i, 16)]
        z = dis_v[sl] * (a_v[sl] + hv) + bv
        o_v[sl] = 1.0 / (1.0 + jnp.exp(-z))
        return 0
    lax.fori_loop(0, ROWS_T // 16, _node, 0)
    pltpu.sync_copy(o_v, out_hbm.at[pl.ds(ROWS_T * s, ROWS_T)])


def kernel(x, edge_index, edge_attr, W1, b1, W2, b2):
    x_p = jnp.pad(x, ((0, NP - N), (0, 0)))
    pad_idx = jnp.arange(EP - E, dtype=jnp.int32) % N
    src_f = jnp.concatenate([edge_index[0], pad_idx])
    dst_f = jnp.concatenate([edge_index[1], pad_idx])
    src = src_f.reshape(NW * NCH, CHUNK)
    dst = dst_f.reshape(NW * NCH, CHUNK)
    w_p = jnp.pad(edge_attr, (0, EP - E))

    d0, d1 = _sc_deg(dst, w_p)
    h1s, dis2 = _tc_b(x_p, W1, d0[:, None], d1[:, None])
    p0, p1 = _sc_rows(src_f, dst_f.reshape(NW * NCH_C, CH_C), w_p, h1s)
    emb_p, h2s2 = _tc_d(p0, p1, h1s, dis2, b1[None, :], W2)
    b2v = jnp.broadcast_to(b2, (16,)).astype(jnp.float32)
    out_flat = _sc_l2(src, dst, w_p, h2s2[:, 0], dis2[:, 0], b2v)
    return (out_flat[:N, None], emb_p[:N])


# speedup vs baseline: 38.0549x; 1.0512x over previous
import functools

import jax
import jax.numpy as jnp
from jax import lax
from jax.experimental import pallas as pl
from jax.experimental.pallas import tpu as pltpu
from jax.experimental.pallas import tpu_sc as plsc

N = 10000
D = 128
E = 320000

NP = 10240
NW = 32
CHUNK = 128
EP_W = 10240
NCH = EP_W // CHUNK
EP = EP_W * NW
ROWS_T = NP // 16

_mesh2 = plsc.VectorSubcoreMesh(core_axis_name="c", subcore_axis_name="s",
                                num_cores=2)
_mesh1 = plsc.VectorSubcoreMesh(core_axis_name="c", subcore_axis_name="s",
                                num_cores=1)


@functools.partial(
    pl.kernel,
    mesh=_mesh2,
    compiler_params=pltpu.CompilerParams(needs_layout_passes=False),
    out_type=[jax.ShapeDtypeStruct((NP,), jnp.float32),
              jax.ShapeDtypeStruct((NP,), jnp.float32)],
    scratch_types=[
        pltpu.VMEM((NCH, CHUNK), jnp.int32),
        pltpu.VMEM((EP_W,), jnp.float32),
        pltpu.VMEM((ROWS_T,), jnp.float32),
        pltpu.SemaphoreType.DMA,
        pltpu.VMEM_SHARED((NP,), jnp.float32),
    ],
)
def _sc_deg(dst_hbm, w_hbm, dp0_hbm, dp1_hbm, dst_v, w_v, stage_v, ssem, acc):
    c = lax.axis_index("c")
    s = lax.axis_index("s")
    wid = c * 16 + s

    def _init(i, _):
        stage_v[pl.ds(16 * i, 16)] = jnp.full((16,), 0.5, jnp.float32)
        return 0
    lax.fori_loop(0, ROWS_T // 16, _init, 0)
    pltpu.sync_copy(stage_v, acc.at[pl.ds(ROWS_T * s, ROWS_T)])
    plsc.subcore_barrier()

    pltpu.sync_copy(dst_hbm.at[pl.ds(wid * NCH, NCH)], dst_v)
    pltpu.sync_copy(w_hbm.at[pl.ds(wid * EP_W, EP_W)], w_v)

    K = 16

    def _chunk(jg, _):
        for u in range(K):
            j = jg * K + u
            pltpu.async_copy(w_v.at[pl.ds(j * CHUNK, CHUNK)],
                             acc.at[dst_v.at[j]], ssem, add=True)
        for u in range(K):
            j = jg * K + u
            pltpu.make_async_copy(w_v.at[pl.ds(j * CHUNK, CHUNK)],
                                  acc.at[dst_v.at[j]], ssem).wait()
        return 0
    lax.fori_loop(0, NCH // K, _chunk, 0)
    plsc.subcore_barrier()

    @pl.when(c == 0)
    def _w0():
        pltpu.sync_copy(acc.at[pl.ds(ROWS_T * s, ROWS_T)],
                        dp0_hbm.at[pl.ds(ROWS_T * s, ROWS_T)])

    @pl.when(c == 1)
    def _w1():
        pltpu.sync_copy(acc.at[pl.ds(ROWS_T * s, ROWS_T)],
                        dp1_hbm.at[pl.ds(ROWS_T * s, ROWS_T)])


def _tc_b_body(x_ref, w1_ref, d0_ref, d1_ref, h1s_ref, dis_ref):
    deg = d0_ref[...] + d1_ref[...]
    dis = lax.rsqrt(deg)
    h1s_ref[...] = jnp.dot(x_ref[...], w1_ref[...],
                           preferred_element_type=jnp.float32) * dis
    dis_ref[...] = dis


RB = 1280


def _tc_b(x_p, W1, d0, d1):
    grid = (NP // RB,)
    return pl.pallas_call(
        _tc_b_body,
        grid=grid,
        in_specs=[
            pl.BlockSpec((RB, D), lambda i: (i, 0)),
            pl.BlockSpec((D, D), lambda i: (0, 0)),
            pl.BlockSpec((RB, 1), lambda i: (i, 0)),
            pl.BlockSpec((RB, 1), lambda i: (i, 0)),
        ],
        out_specs=[
            pl.BlockSpec((RB, D), lambda i: (i, 0)),
            pl.BlockSpec((RB, 1), lambda i: (i, 0)),
        ],
        out_shape=[
            jax.ShapeDtypeStruct((NP, D), jnp.float32),
            jax.ShapeDtypeStruct((NP, 1), jnp.float32),
        ],
    )(x_p, W1, d0, d1)


CH_C = 64
NCH_C = EP_W // CH_C


@functools.partial(
    pl.kernel,
    mesh=_mesh2,
    compiler_params=pltpu.CompilerParams(needs_layout_passes=False),
    out_type=[jax.ShapeDtypeStruct((NP, D), jnp.float32),
              jax.ShapeDtypeStruct((NP, D), jnp.float32)],
    scratch_types=[
        pltpu.VMEM((EP_W,), jnp.int32),
        pltpu.VMEM((NCH_C, CH_C), jnp.int32),
        pltpu.VMEM((CH_C,), jnp.float32),
        pltpu.VMEM((CH_C,), jnp.float32),
        pltpu.VMEM((CH_C, D), jnp.float32),
        pltpu.VMEM((CH_C, D), jnp.float32),
        pltpu.SemaphoreType.DMA,
        pltpu.SemaphoreType.DMA,
        pltpu.SemaphoreType.DMA,
        pltpu.SemaphoreType.DMA,
        pltpu.SemaphoreType.DMA,
        pltpu.SemaphoreType.DMA,
        pltpu.VMEM_SHARED((NP, D), jnp.float32),
    ],
)
def _sc_rows(src_hbm, dst_hbm, w_hbm, h1s_hbm, p0_hbm, p1_hbm,
             src_v, dst_v, wb0, wb1, rows0, rows1,
             g0, g1, gw0, gw1, s0, s1, acc):
    c = lax.axis_index("c")
    s = lax.axis_index("s")
    wid = c * 16 + s

    def _zrow(i, _):
        for k in range(D // 16):
            rows0[i, pl.ds(16 * k, 16)] = jnp.zeros((16,), jnp.float32)
        return 0
    lax.fori_loop(0, CH_C, _zrow, 0)
    for m in range(ROWS_T // CH_C):
        pltpu.sync_copy(rows0, acc.at[pl.ds(ROWS_T * s + CH_C * m, CH_C)])
    plsc.subcore_barrier()

    pltpu.sync_copy(src_hbm.at[pl.ds(wid * EP_W, EP_W)], src_v)
    pltpu.sync_copy(dst_hbm.at[pl.ds(wid * NCH_C, NCH_C)], dst_v)

    def _issue(n, rowsb, wbuf, gs, gws):
        pltpu.async_copy(h1s_hbm.at[src_v.at[pl.ds(n * CH_C, CH_C)]],
                         rowsb, gs)
        pltpu.async_copy(w_hbm.at[pl.ds(wid * EP_W + n * CH_C, CH_C)],
                         wbuf, gws)

    def _wait(n, rowsb, wbuf, gs, gws):
        pltpu.make_async_copy(h1s_hbm.at[src_v.at[pl.ds(n * CH_C, CH_C)]],
                              rowsb, gs).wait()
        pltpu.make_async_copy(w_hbm.at[pl.ds(wid * EP_W + n * CH_C, CH_C)],
                              wbuf, gws).wait()

    def _scale(buf, wbuf):
        for m in range(CH_C // 16):
            w16 = wbuf[pl.ds(16 * m, 16)]
            for t in range(16):
                wv = jnp.take_along_axis(w16, jnp.full((16,), t, jnp.int32),
                                         axis=0)
                e = 16 * m + t
                for k in range(D // 16):
                    buf[e, pl.ds(16 * k, 16)] = buf[e, pl.ds(16 * k, 16)] * wv

    _issue(0, rows0, wb0, g0, gw0)
    _issue(1, rows1, wb1, g1, gw1)

    def _pair(t, _):
        c0 = 2 * t
        c1 = 2 * t + 1
        _wait(c0, rows0, wb0, g0, gw0)
        _scale(rows0, wb0)
        cp0 = pltpu.async_copy(rows0, acc.at[dst_v.at[c0]], s0, add=True)
        _wait(c1, rows1, wb1, g1, gw1)
        _scale(rows1, wb1)
        cp1 = pltpu.async_copy(rows1, acc.at[dst_v.at[c1]], s1, add=True)
        n0 = jnp.minimum(c0 + 2, NCH_C - 1)
        n1 = jnp.minimum(c1 + 2, NCH_C - 1)
        cp0.wait()
        _issue(n0, rows0, wb0, g0, gw0)
        cp1.wait()
        _issue(n1, rows1, wb1, g1, gw1)
        return 0
    lax.fori_loop(0, NCH_C // 2, _pair, 0)
    _wait(NCH_C - 1, rows0, wb0, g0, gw0)
    _wait(NCH_C - 1, rows1, wb1, g1, gw1)
    plsc.subcore_barrier()

    for m in range(ROWS_T // CH_C):
        r = ROWS_T * s + CH_C * m
        @pl.when(c == 0)
        def _w0():
            pltpu.sync_copy(acc.at[pl.ds(r, CH_C)], p0_hbm.at[pl.ds(r, CH_C)])

        @pl.when(c == 1)
        def _w1():
            pltpu.sync_copy(acc.at[pl.ds(r, CH_C)], p1_hbm.at[pl.ds(r, CH_C)])


def _tc_d_body(p0_ref, p1_ref, h1s_ref, dis_ref, b1_ref, w2_ref,
               emb_ref, h2s_ref):
    dis = dis_ref[...]
    s = dis * (p0_ref[...] + p1_ref[...] + h1s_ref[...]) + b1_ref[...]
    h = jnp.where(s > 0, s, jnp.exp(s) - 1.0)
    emb_ref[...] = h
    h2s_ref[...] = jnp.dot(h * dis, w2_ref[...],
                           preferred_element_type=jnp.float32)


def _tc_d(p0, p1, h1s, dis2, b1r, W2):
    grid = (NP // RB,)
    return pl.pallas_call(
        _tc_d_body,
        grid=grid,
        in_specs=[
            pl.BlockSpec((RB, D), lambda i: (i, 0)),
            pl.BlockSpec((RB, D), lambda i: (i, 0)),
            pl.BlockSpec((RB, D), lambda i: (i, 0)),
            pl.BlockSpec((RB, 1), lambda i: (i, 0)),
            pl.BlockSpec((1, D), lambda i: (0, 0)),
            pl.BlockSpec((D, 1), lambda i: (0, 0)),
        ],
        out_specs=[
            pl.BlockSpec((RB, D), lambda i: (i, 0)),
            pl.BlockSpec((RB, 1), lambda i: (i, 0)),
        ],
        out_shape=[
            jax.ShapeDtypeStruct((NP, D), jnp.float32),
            jax.ShapeDtypeStruct((NP, 1), jnp.float32),
        ],
    )(p0, p1, h1s, dis2, b1r, W2)


EP_W1 = EP // 16
NCH1 = EP_W1 // CHUNK


@functools.partial(
    pl.kernel,
    mesh=_mesh1,
    compiler_params=pltpu.CompilerParams(needs_layout_passes=False),
    out_type=jax.ShapeDtypeStruct((NP,), jnp.float32),
    scratch_types=[
        pltpu.VMEM((NCH1, CHUNK), jnp.int32),
        pltpu.VMEM((NCH1, CHUNK), jnp.int32),
        pltpu.VMEM((EP_W1,), jnp.float32),
        pltpu.VMEM((NP,), jnp.float32),
        pltpu.VMEM((CHUNK,), jnp.float32),
        pltpu.VMEM((CHUNK,), jnp.float32),
        pltpu.VMEM((ROWS_T,), jnp.float32),
        pltpu.VMEM((ROWS_T,), jnp.float32),
        pltpu.VMEM((ROWS_T,), jnp.float32),
        pltpu.VMEM((16,), jnp.float32),
        pltpu.SemaphoreType.DMA,
        pltpu.SemaphoreType.DMA,
        pltpu.VMEM_SHARED((NP,), jnp.float32),
    ],
)
def _sc_l2(src_hbm, dst_hbm, w_hbm, h2s_hbm, dis_hbm, b2_hbm, out_hbm,
           src_v, dst_v, w_v, h2s_v, msg0, msg1, a_v, dis_v, o_v, b2_v,
           se0, se1, acc):
    s = lax.axis_index("s")

    def _init(i, _):
        o_v[pl.ds(16 * i, 16)] = jnp.zeros((16,), jnp.float32)
        return 0
    lax.fori_loop(0, ROWS_T // 16, _init, 0)
    pltpu.sync_copy(o_v, acc.at[pl.ds(ROWS_T * s, ROWS_T)])
    plsc.subcore_barrier()

    pltpu.sync_copy(src_hbm.at[pl.ds(s * NCH1, NCH1)], src_v)
    pltpu.sync_copy(dst_hbm.at[pl.ds(s * NCH1, NCH1)], dst_v)
    pltpu.sync_copy(w_hbm.at[pl.ds(s * EP_W1, EP_W1)], w_v)
    pltpu.sync_copy(h2s_hbm, h2s_v)

    def _msgs(mb, j):
        for m in range(CHUNK // 16):
            idx = src_v[j, pl.ds(16 * m, 16)]
            hv = plsc.load_gather(h2s_v, [idx])
            wv = w_v[pl.ds(j * CHUNK + 16 * m, 16)]
            mb[pl.ds(16 * m, 16)] = hv * wv

    def _pair(t, _):
        j0 = 2 * t
        j1 = j0 + 1

        @pl.when(t > 0)
        def _d0():
            pltpu.make_async_copy(msg0, acc.at[dst_v.at[j0 - 2]], se0).wait()
        _msgs(msg0, j0)
        pltpu.async_copy(msg0, acc.at[dst_v.at[j0]], se0, add=True)

        @pl.when(t > 0)
        def _d1():
            pltpu.make_async_copy(msg1, acc.at[dst_v.at[j1 - 2]], se1).wait()
        _msgs(msg1, j1)
        pltpu.async_copy(msg1, acc.at[dst_v.at[j1]], se1, add=True)
        return 0
    lax.fori_loop(0, NCH1 // 2, _pair, 0)
    pltpu.make_async_copy(msg0, acc.at[dst_v.at[NCH1 - 2]], se0).wait()
    pltpu.make_async_copy(msg1, acc.at[dst_v.at[NCH1 - 1]], se1).wait()
    plsc.subcore_barrier()

    pltpu.sync_copy(acc.at[pl.ds(ROWS_T * s, ROWS_T)], a_v)
    pltpu.sync_copy(dis_hbm.at[pl.ds(ROWS_T * s, ROWS_T)], dis_v)
    pltpu.sync_copy(b2_hbm, b2_v)
    bv = b2_v[...]

    def _node(i, _):
        sl = pl.ds(16 * i, 16)
        hv = h2s_v[pl.ds(ROWS_T * s + 16 * i, 16)]
        z = dis_v[sl] * (a_v[sl] + hv) + bv
        o_v[sl] = 1.0 / (1.0 + jnp.exp(-z))
        return 0
    lax.fori_loop(0, ROWS_T // 16, _node, 0)
    pltpu.sync_copy(o_v, out_hbm.at[pl.ds(ROWS_T * s, ROWS_T)])


def kernel(x, edge_index, edge_attr, W1, b1, W2, b2):
    pad_idx = jnp.arange(EP - E, dtype=jnp.int32) % N
    src_f = jnp.concatenate([edge_index[0], pad_idx])
    dst_f = jnp.concatenate([edge_index[1], pad_idx])
    src = src_f.reshape(NW * NCH, CHUNK)
    dst = dst_f.reshape(NW * NCH, CHUNK)
    w_p = jnp.pad(edge_attr, (0, EP - E))

    d0, d1 = _sc_deg(dst, w_p)
    h1s, dis2 = _tc_b(x, W1, d0[:, None], d1[:, None])
    p0, p1 = _sc_rows(src_f, dst_f.reshape(NW * NCH_C, CH_C), w_p, h1s)
    emb_p, h2s2 = _tc_d(p0, p1, h1s, dis2, b1[None, :], W2)
    b2v = jnp.broadcast_to(b2, (16,)).astype(jnp.float32)
    out_flat = _sc_l2(src, dst, w_p, h2s2[:, 0], dis2[:, 0], b2v)
    return (out_flat[:N, None], emb_p[:N])


# speedup vs baseline: 38.1484x; 1.0025x over previous
import functools

import numpy as np

import jax
import jax.numpy as jnp
from jax import lax
from jax.experimental import pallas as pl
from jax.experimental.pallas import tpu as pltpu
from jax.experimental.pallas import tpu_sc as plsc

N = 10000
D = 128
E = 320000

NP = 10240
NW = 32
CHUNK = 128
EP_W = 10240
NCH = EP_W // CHUNK
EP = EP_W * NW
ROWS_T = NP // 16

_mesh2 = plsc.VectorSubcoreMesh(core_axis_name="c", subcore_axis_name="s",
                                num_cores=2)
_mesh1 = plsc.VectorSubcoreMesh(core_axis_name="c", subcore_axis_name="s",
                                num_cores=1)


@functools.partial(
    pl.kernel,
    mesh=_mesh2,
    compiler_params=pltpu.CompilerParams(needs_layout_passes=False),
    out_type=[jax.ShapeDtypeStruct((NP,), jnp.float32),
              jax.ShapeDtypeStruct((NP,), jnp.float32)],
    scratch_types=[
        pltpu.VMEM((NCH, CHUNK), jnp.int32),
        pltpu.VMEM((EP_W,), jnp.float32),
        pltpu.VMEM((ROWS_T,), jnp.float32),
        pltpu.SemaphoreType.DMA,
        pltpu.VMEM_SHARED((NP,), jnp.float32),
    ],
)
def _sc_deg(dst_hbm, w_hbm, dp0_hbm, dp1_hbm, dst_v, w_v, stage_v, ssem, acc):
    c = lax.axis_index("c")
    s = lax.axis_index("s")
    wid = c * 16 + s

    def _init(i, _):
        stage_v[pl.ds(16 * i, 16)] = jnp.full((16,), 0.5, jnp.float32)
        return 0
    lax.fori_loop(0, ROWS_T // 16, _init, 0)
    pltpu.sync_copy(stage_v, acc.at[pl.ds(ROWS_T * s, ROWS_T)])
    plsc.subcore_barrier()

    pltpu.sync_copy(dst_hbm.at[pl.ds(wid * NCH, NCH)], dst_v)
    pltpu.sync_copy(w_hbm.at[pl.ds(wid * EP_W, EP_W)], w_v)

    K = 16

    def _chunk(jg, _):
        for u in range(K):
            j = jg * K + u
            pltpu.async_copy(w_v.at[pl.ds(j * CHUNK, CHUNK)],
                             acc.at[dst_v.at[j]], ssem, add=True)
        for u in range(K):
            j = jg * K + u
            pltpu.make_async_copy(w_v.at[pl.ds(j * CHUNK, CHUNK)],
                                  acc.at[dst_v.at[j]], ssem).wait()
        return 0
    lax.fori_loop(0, NCH // K, _chunk, 0)
    plsc.subcore_barrier()

    @pl.when(c == 0)
    def _w0():
        pltpu.sync_copy(acc.at[pl.ds(ROWS_T * s, ROWS_T)],
                        dp0_hbm.at[pl.ds(ROWS_T * s, ROWS_T)])

    @pl.when(c == 1)
    def _w1():
        pltpu.sync_copy(acc.at[pl.ds(ROWS_T * s, ROWS_T)],
                        dp1_hbm.at[pl.ds(ROWS_T * s, ROWS_T)])


def _tc_b_body(x_ref, w1_ref, d0_ref, d1_ref, h1s_ref, dis_ref):
    deg = d0_ref[...] + d1_ref[...]
    dis = lax.rsqrt(deg)
    h1s_ref[...] = jnp.dot(x_ref[...], w1_ref[...],
                           preferred_element_type=jnp.float32) * dis
    dis_ref[...] = dis


RB = 1280


def _tc_b(x_p, W1, d0, d1):
    grid = (NP // RB,)
    return pl.pallas_call(
        _tc_b_body,
        grid=grid,
        in_specs=[
            pl.BlockSpec((RB, D), lambda i: (i, 0)),
            pl.BlockSpec((D, D), lambda i: (0, 0)),
            pl.BlockSpec((RB, 1), lambda i: (i, 0)),
            pl.BlockSpec((RB, 1), lambda i: (i, 0)),
        ],
        out_specs=[
            pl.BlockSpec((RB, D), lambda i: (i, 0)),
            pl.BlockSpec((RB, 1), lambda i: (i, 0)),
        ],
        out_shape=[
            jax.ShapeDtypeStruct((NP, D), jnp.float32),
            jax.ShapeDtypeStruct((NP, 1), jnp.float32),
        ],
    )(x_p, W1, d0, d1)


CH_C = 64
NCH_C = EP_W // CH_C


@functools.partial(
    pl.kernel,
    mesh=_mesh2,
    compiler_params=pltpu.CompilerParams(needs_layout_passes=False),
    out_type=[jax.ShapeDtypeStruct((NP, D), jnp.float32),
              jax.ShapeDtypeStruct((NP, D), jnp.float32)],
    scratch_types=[
        pltpu.VMEM((EP_W,), jnp.int32),
        pltpu.VMEM((NCH_C, CH_C), jnp.int32),
        pltpu.VMEM((CH_C,), jnp.float32),
        pltpu.VMEM((CH_C,), jnp.float32),
        pltpu.VMEM((CH_C, D), jnp.float32),
        pltpu.VMEM((CH_C, D), jnp.float32),
        pltpu.SemaphoreType.DMA,
        pltpu.SemaphoreType.DMA,
        pltpu.SemaphoreType.DMA,
        pltpu.SemaphoreType.DMA,
        pltpu.SemaphoreType.DMA,
        pltpu.SemaphoreType.DMA,
        pltpu.VMEM_SHARED((NP, D), jnp.float32),
    ],
)
def _sc_rows(src_hbm, dst_hbm, w_hbm, h1s_hbm, p0_hbm, p1_hbm,
             src_v, dst_v, wb0, wb1, rows0, rows1,
             g0, g1, gw0, gw1, s0, s1, acc):
    c = lax.axis_index("c")
    s = lax.axis_index("s")
    wid = c * 16 + s

    def _zrow(i, _):
        for k in range(D // 16):
            rows0[i, pl.ds(16 * k, 16)] = jnp.zeros((16,), jnp.float32)
        return 0
    lax.fori_loop(0, CH_C, _zrow, 0)
    for m in range(ROWS_T // CH_C):
        pltpu.sync_copy(rows0, acc.at[pl.ds(ROWS_T * s + CH_C * m, CH_C)])
    plsc.subcore_barrier()

    pltpu.sync_copy(src_hbm.at[pl.ds(wid * EP_W, EP_W)], src_v)
    pltpu.sync_copy(dst_hbm.at[pl.ds(wid * NCH_C, NCH_C)], dst_v)

    def _issue(n, rowsb, wbuf, gs, gws):
        pltpu.async_copy(h1s_hbm.at[src_v.at[pl.ds(n * CH_C, CH_C)]],
                         rowsb, gs)
        pltpu.async_copy(w_hbm.at[pl.ds(wid * EP_W + n * CH_C, CH_C)],
                         wbuf, gws)

    def _wait(n, rowsb, wbuf, gs, gws):
        pltpu.make_async_copy(h1s_hbm.at[src_v.at[pl.ds(n * CH_C, CH_C)]],
                              rowsb, gs).wait()
        pltpu.make_async_copy(w_hbm.at[pl.ds(wid * EP_W + n * CH_C, CH_C)],
                              wbuf, gws).wait()

    def _scale(buf, wbuf):
        for m in range(CH_C // 16):
            w16 = wbuf[pl.ds(16 * m, 16)]
            for t in range(16):
                wv = jnp.take_along_axis(w16, jnp.full((16,), t, jnp.int32),
                                         axis=0)
                e = 16 * m + t
                for k in range(D // 16):
                    buf[e, pl.ds(16 * k, 16)] = buf[e, pl.ds(16 * k, 16)] * wv

    _issue(0, rows0, wb0, g0, gw0)
    _issue(1, rows1, wb1, g1, gw1)

    def _pair(t, _):
        c0 = 2 * t
        c1 = 2 * t + 1
        _wait(c0, rows0, wb0, g0, gw0)
        _scale(rows0, wb0)
        cp0 = pltpu.async_copy(rows0, acc.at[dst_v.at[c0]], s0, add=True)
        _wait(c1, rows1, wb1, g1, gw1)
        _scale(rows1, wb1)
        cp1 = pltpu.async_copy(rows1, acc.at[dst_v.at[c1]], s1, add=True)
        n0 = jnp.minimum(c0 + 2, NCH_C - 1)
        n1 = jnp.minimum(c1 + 2, NCH_C - 1)
        cp0.wait()
        _issue(n0, rows0, wb0, g0, gw0)
        cp1.wait()
        _issue(n1, rows1, wb1, g1, gw1)
        return 0
    lax.fori_loop(0, NCH_C // 2, _pair, 0)
    _wait(NCH_C - 1, rows0, wb0, g0, gw0)
    _wait(NCH_C - 1, rows1, wb1, g1, gw1)
    plsc.subcore_barrier()

    for m in range(ROWS_T // CH_C):
        r = ROWS_T * s + CH_C * m
        @pl.when(c == 0)
        def _w0():
            pltpu.sync_copy(acc.at[pl.ds(r, CH_C)], p0_hbm.at[pl.ds(r, CH_C)])

        @pl.when(c == 1)
        def _w1():
            pltpu.sync_copy(acc.at[pl.ds(r, CH_C)], p1_hbm.at[pl.ds(r, CH_C)])


def _tc_d_body(p0_ref, p1_ref, h1s_ref, dis_ref, b1_ref, w2_ref,
               emb_ref, h2s_ref):
    dis = dis_ref[...]
    s = dis * (p0_ref[...] + p1_ref[...] + h1s_ref[...]) + b1_ref[...]
    h = jnp.where(s > 0, s, jnp.exp(s) - 1.0)
    emb_ref[...] = h
    h2s_ref[...] = jnp.dot(h * dis, w2_ref[...],
                           preferred_element_type=jnp.float32)


def _tc_d(p0, p1, h1s, dis2, b1r, W2):
    grid = (NP // RB,)
    return pl.pallas_call(
        _tc_d_body,
        grid=grid,
        in_specs=[
            pl.BlockSpec((RB, D), lambda i: (i, 0)),
            pl.BlockSpec((RB, D), lambda i: (i, 0)),
            pl.BlockSpec((RB, D), lambda i: (i, 0)),
            pl.BlockSpec((RB, 1), lambda i: (i, 0)),
            pl.BlockSpec((1, D), lambda i: (0, 0)),
            pl.BlockSpec((D, 1), lambda i: (0, 0)),
        ],
        out_specs=[
            pl.BlockSpec((RB, D), lambda i: (i, 0)),
            pl.BlockSpec((RB, 1), lambda i: (i, 0)),
        ],
        out_shape=[
            jax.ShapeDtypeStruct((N, D), jnp.float32),
            jax.ShapeDtypeStruct((NP, 1), jnp.float32),
        ],
    )(p0, p1, h1s, dis2, b1r, W2)


EP_W1 = EP // 16
NCH1 = EP_W1 // CHUNK


@functools.partial(
    pl.kernel,
    mesh=_mesh1,
    compiler_params=pltpu.CompilerParams(needs_layout_passes=False),
    out_type=jax.ShapeDtypeStruct((NP,), jnp.float32),
    scratch_types=[
        pltpu.VMEM((EP_W1,), jnp.int32),
        pltpu.VMEM((NCH1, CHUNK), jnp.int32),
        pltpu.VMEM((EP_W1,), jnp.float32),
        pltpu.VMEM((NP,), jnp.float32),
        pltpu.VMEM((CHUNK,), jnp.float32),
        pltpu.VMEM((CHUNK,), jnp.float32),
        pltpu.VMEM((ROWS_T,), jnp.float32),
        pltpu.VMEM((ROWS_T,), jnp.float32),
        pltpu.VMEM((ROWS_T,), jnp.float32),
        pltpu.VMEM((16,), jnp.float32),
        pltpu.SemaphoreType.DMA,
        pltpu.SemaphoreType.DMA,
        pltpu.VMEM_SHARED((NP,), jnp.float32),
    ],
)
def _sc_l2(src_hbm, dst_hbm, w_hbm, h2s_hbm, dis_hbm, b2_hbm, out_hbm,
           src_v, dst_v, w_v, h2s_v, msg0, msg1, a_v, dis_v, o_v, b2_v,
           se0, se1, acc):
    s = lax.axis_index("s")

    def _init(i, _):
        o_v[pl.ds(16 * i, 16)] = jnp.zeros((16,), jnp.float32)
        return 0
    lax.fori_loop(0, ROWS_T // 16, _init, 0)
    pltpu.sync_copy(o_v, acc.at[pl.ds(ROWS_T * s, ROWS_T)])
    plsc.subcore_barrier()

    pltpu.sync_copy(src_hbm.at[pl.ds(s * EP_W1, EP_W1)], src_v)
    pltpu.sync_copy(dst_hbm.at[pl.ds(s * NCH1, NCH1)], dst_v)
    pltpu.sync_copy(w_hbm.at[pl.ds(s * EP_W1, EP_W1)], w_v)
    pltpu.sync_copy(h2s_hbm, h2s_v)

    def _msgs(mb, j):
        for m in range(CHUNK // 16):
            idx = src_v[pl.ds(j * CHUNK + 16 * m, 16)]
            hv = plsc.load_gather(h2s_v, [idx])
            wv = w_v[pl.ds(j * CHUNK + 16 * m, 16)]
            mb[pl.ds(16 * m, 16)] = hv * wv

    def _pair(t, _):
        j0 = 2 * t
        j1 = j0 + 1

        @pl.when(t > 0)
        def _d0():
            pltpu.make_async_copy(msg0, acc.at[dst_v.at[j0 - 2]], se0).wait()
        _msgs(msg0, j0)
        pltpu.async_copy(msg0, acc.at[dst_v.at[j0]], se0, add=True)

        @pl.when(t > 0)
        def _d1():
            pltpu.make_async_copy(msg1, acc.at[dst_v.at[j1 - 2]], se1).wait()
        _msgs(msg1, j1)
        pltpu.async_copy(msg1, acc.at[dst_v.at[j1]], se1, add=True)
        return 0
    lax.fori_loop(0, NCH1 // 2, _pair, 0)
    pltpu.make_async_copy(msg0, acc.at[dst_v.at[NCH1 - 2]], se0).wait()
    pltpu.make_async_copy(msg1, acc.at[dst_v.at[NCH1 - 1]], se1).wait()
    plsc.subcore_barrier()

    pltpu.sync_copy(acc.at[pl.ds(ROWS_T * s, ROWS_T)], a_v)
    pltpu.sync_copy(dis_hbm.at[pl.ds(ROWS_T * s, ROWS_T)], dis_v)
    pltpu.sync_copy(b2_hbm, b2_v)
    bv = b2_v[...]

    def _node(i, _):
        sl = pl.ds(16 * i, 16)
        hv = h2s_v[pl.ds(ROWS_T * s + 16 * i, 16)]
        z = dis_v[sl] * (a_v[sl] + hv) + bv
        o_v[sl] = 1.0 / (1.0 + jnp.exp(-z))
        return 0
    lax.fori_loop(0, ROWS_T // 16, _node, 0)
    pltpu.sync_copy(o_v, out_hbm.at[pl.ds(ROWS_T * s, ROWS_T)])


def kernel(x, edge_index, edge_attr, W1, b1, W2, b2):
    pad_idx = jnp.asarray(np.arange(EP - E, dtype=np.int32))
    src_f = jnp.concatenate([edge_index[0], pad_idx])
    dst_f = jnp.concatenate([edge_index[1], pad_idx])
    dst = dst_f.reshape(NW * NCH, CHUNK)
    w_p = jnp.pad(edge_attr, (0, EP - E))

    d0, d1 = _sc_deg(dst, w_p)
    h1s, dis2 = _tc_b(x, W1, d0[:, None], d1[:, None])
    p0, p1 = _sc_rows(src_f, dst_f.reshape(NW * NCH_C, CH_C), w_p, h1s)
    emb_p, h2s2 = _tc_d(p0, p1, h1s, dis2, b1[None, :], W2)
    b2v = jnp.broadcast_to(b2, (16,)).astype(jnp.float32)
    out_flat = _sc_l2(src_f, dst, w_p, h2s2[:, 0], dis2[:, 0], b2v)
    return (out_flat[:N, None], emb_p)
